# baseline (speedup 1.0000x reference)
import jax
import jax.numpy as jnp
from jax import lax
from jax.experimental import pallas as pl
from jax.experimental.pallas import tpu as pltpu
from jax.experimental.pallas import tpu_sc as plsc

N = 10000
E = 320000
D = 128
H = 128
ED = 16
XC = 4
PW = 2 * H

NC = 2
NS = 16
NW = NC * NS
L = 16
WIN = 40

CH = 2
EC = E // CH

ROWS_A = 632
ROWS_B = N - (NS - 1) * ROWS_A

_mesh = plsc.VectorSubcoreMesh(
    core_axis_name="c", subcore_axis_name="s", num_cores=NC, num_subcores=NS)

_f32 = jnp.float32


def _sig(t):
  return 1.0 / (1.0 + jnp.exp(-t))


def _proj_body(h_ref, wsrc_ref, wdst_ref, psrc_ref, pdst_ref):
  hb = h_ref[:]
  psrc_ref[:] = jnp.dot(hb, wsrc_ref[:], preferred_element_type=_f32)
  pdst_ref[:] = jnp.dot(hb, wdst_ref[:], preferred_element_type=_f32)


_proj = pl.pallas_call(
    _proj_body,
    out_shape=[jax.ShapeDtypeStruct((N, PW), _f32),
               jax.ShapeDtypeStruct((N, PW), _f32)],
)


def _make_gather(ne):
  epw = ne // NW
  nwin = epw // WIN

  def body(src_ref, dst_ref, psrc_ref, pdst_ref,
           z0_ref,
           idxs_v, idxd_v, gs0, gd0, gs1, gd1,
           sem_a0, sem_b0, sem_a1, sem_b1):
    wid = lax.axis_index("s") * NC + lax.axis_index("c")
    base = wid * epw

    pltpu.sync_copy(src_ref.at[pl.ds(base, epw)], idxs_v)
    pltpu.sync_copy(dst_ref.at[pl.ds(base, epw)], idxd_v)

    def issue(w, gs, gd, sa, sb):
      sl = pl.ds(w * WIN, WIN)
      pltpu.async_copy(psrc_ref.at[idxs_v.at[sl]], gs, sa)
      pltpu.async_copy(pdst_ref.at[idxd_v.at[sl]], gd, sb)

    def process(w, gs, gd, sa, sb):
      pltpu.make_async_copy(
          psrc_ref.at[idxs_v.at[pl.ds(0, WIN)]], gs, sa).wait()
      pltpu.make_async_copy(
          pdst_ref.at[idxd_v.at[pl.ds(0, WIN)]], gd, sb).wait()

      def row(i, c2):
        for k in range(PW // L):
          sl = pl.ds(k * L, L)
          gs[i, sl] = gs[i, sl] + gd[i, sl]
        return c2

      lax.fori_loop(0, WIN, row, 0)
      pltpu.sync_copy(gs, z0_ref.at[pl.ds(base + w * WIN, WIN)])

    issue(0, gs0, gd0, sem_a0, sem_b0)

    def pair(k, carry):
      w0 = 2 * k
      issue(w0 + 1, gs1, gd1, sem_a1, sem_b1)
      process(w0, gs0, gd0, sem_a0, sem_b0)

      @pl.when(w0 + 2 < nwin)
      def _():
        issue(w0 + 2, gs0, gd0, sem_a0, sem_b0)

      process(w0 + 1, gs1, gd1, sem_a1, sem_b1)
      return carry

    lax.fori_loop(0, nwin // 2, pair, 0)
    if nwin % 2 == 1:
      process(nwin - 1, gs0, gd0, sem_a0, sem_b0)

  return pl.kernel(
      body,
      out_type=jax.ShapeDtypeStruct((ne, PW), _f32),
      mesh=_mesh,
      scratch_types=[
          pltpu.VMEM((epw,), jnp.int32),
          pltpu.VMEM((epw,), jnp.int32),
          pltpu.VMEM((WIN, PW), _f32),
          pltpu.VMEM((WIN, PW), _f32),
          pltpu.VMEM((WIN, PW), _f32),
          pltpu.VMEM((WIN, PW), _f32),
          pltpu.SemaphoreType.DMA,
          pltpu.SemaphoreType.DMA,
          pltpu.SemaphoreType.DMA,
          pltpu.SemaphoreType.DMA,
      ],
      cost_estimate=pl.CostEstimate(
          flops=ne * PW, transcendentals=0,
          bytes_accessed=3 * ne * PW * 4),
  )


BE = 3200


def _make_edge(ne):
  def body(dist_ref, z0_ref,
           we1_ref, be1_ref, we2_ref, be2_ref,
           vcat_ref, bcat_ref, wc2_ref,
           sv_ref, w_ref):
    i = pl.program_id(0)
    d = dist_ref[pl.ds(i * BE, BE)]
    e1 = d[:, None] * we1_ref[:] + be1_ref[:]
    e1 = e1 * _sig(e1)
    e2 = jnp.dot(e1, we2_ref[:], preferred_element_type=_f32) + be2_ref[:]
    z = z0_ref[:] + jnp.dot(e2, vcat_ref[:], preferred_element_type=_f32)
    z = z + bcat_ref[:]
    s = z * _sig(z)
    sv_ref[:] = s[:, :H]
    s_c3 = s[:, H:].reshape(BE // 128, 128, H)
    wc2b = jnp.broadcast_to(wc2_ref[:].reshape(1, 1, H), (BE // 128, 1, H))
    wt = lax.dot_general(wc2b, s_c3, (((2,), (2,)), ((0,), (0,))),
                         preferred_element_type=_f32)
    w_ref[:] = wt.reshape(1, BE // 128, 128)

  return pl.pallas_call(
      body,
      grid=(ne // BE,),
      in_specs=[
          pl.BlockSpec((ne,), lambda i: (0,)),
          pl.BlockSpec((BE, PW), lambda i: (i, 0)),
          pl.BlockSpec((1, ED), lambda i: (0, 0)),
          pl.BlockSpec((1, ED), lambda i: (0, 0)),
          pl.BlockSpec((ED, ED), lambda i: (0, 0)),
          pl.BlockSpec((1, ED), lambda i: (0, 0)),
          pl.BlockSpec((ED, PW), lambda i: (0, 0)),
          pl.BlockSpec((1, PW), lambda i: (0, 0)),
          pl.BlockSpec((1, H), lambda i: (0, 0)),
      ],
      out_specs=[
          pl.BlockSpec((BE, H), lambda i: (i, 0)),
          pl.BlockSpec((1, BE // 128, 128), lambda i: (i, 0, 0)),
      ],
      out_shape=[jax.ShapeDtypeStruct((ne, H), _f32),
                 jax.ShapeDtypeStruct((ne // BE, BE // 128, 128), _f32)],
      compiler_params=pltpu.CompilerParams(
          dimension_semantics=("arbitrary",)),
  )


def _make_scatter_h(ne):
  epw = ne // NW
  nwin = epw // WIN

  def body(dst_ref, sv_hbm, zh_ref, hpart_ref,
           idxd0, idxd1, sv0, sv1, acc_s,
           sem_i0, sem_s0, sem_i1, sem_s1):
    cid = lax.axis_index("c")
    sid = lax.axis_index("s")
    wid = sid * NC + cid
    base = wid * epw
    row0 = sid * ROWS_A

    @pl.when(sid < NS - 1)
    def _():
      pltpu.sync_copy(zh_ref.at[pl.ds(row0, ROWS_A)],
                      acc_s.at[pl.ds(row0, ROWS_A)])

    @pl.when(sid == NS - 1)
    def _():
      pltpu.sync_copy(zh_ref.at[pl.ds((NS - 1) * ROWS_A, ROWS_B)],
                      acc_s.at[pl.ds((NS - 1) * ROWS_A, ROWS_B)])

    plsc.subcore_barrier()

    def issue(w, idx_v, sv_v, si, ss):
      off = base + w * WIN
      pltpu.async_copy(dst_ref.at[pl.ds(off, WIN)], idx_v, si)
      pltpu.async_copy(sv_hbm.at[pl.ds(off, WIN)], sv_v, ss)

    def process(idx_v, sv_v, si, ss):
      pltpu.make_async_copy(dst_ref.at[pl.ds(0, WIN)], idx_v, si).wait()
      pltpu.make_async_copy(sv_hbm.at[pl.ds(0, WIN)], sv_v, ss).wait()
      pltpu.sync_copy(sv_v, acc_s.at[idx_v], add=True)

    issue(0, idxd0, sv0, sem_i0, sem_s0)

    def pair(k, carry):
      issue(2 * k + 1, idxd1, sv1, sem_i1, sem_s1)
      process(idxd0, sv0, sem_i0, sem_s0)

      @pl.when(2 * k + 2 < nwin)
      def _():
        issue(2 * k + 2, idxd0, sv0, sem_i0, sem_s0)

      process(idxd1, sv1, sem_i1, sem_s1)
      return carry

    lax.fori_loop(0, nwin // 2, pair, 0)
    if nwin % 2 == 1:
      process(idxd0, sv0, sem_i0, sem_s0)

    plsc.subcore_barrier()

    @pl.when(sid < NS - 1)
    def _():
      pltpu.sync_copy(acc_s.at[pl.ds(row0, ROWS_A)],
                      hpart_ref.at[cid, pl.ds(row0, ROWS_A)])

    @pl.when(sid == NS - 1)
    def _():
      pltpu.sync_copy(acc_s.at[pl.ds((NS - 1) * ROWS_A, ROWS_B)],
                      hpart_ref.at[cid, pl.ds((NS - 1) * ROWS_A, ROWS_B)])

  return pl.kernel(
      body,
      out_type=jax.ShapeDtypeStruct((NC, N, H), _f32),
      mesh=_mesh,
      scratch_types=[
          pltpu.VMEM((WIN,), jnp.int32),
          pltpu.VMEM((WIN,), jnp.int32),
          pltpu.VMEM((WIN, H), _f32),
          pltpu.VMEM((WIN, H), _f32),
          pltpu.VMEM_SHARED((N, H), _f32),
          pltpu.SemaphoreType.DMA,
          pltpu.SemaphoreType.DMA,
          pltpu.SemaphoreType.DMA,
          pltpu.SemaphoreType.DMA,
      ],
      cost_estimate=pl.CostEstimate(
          flops=ne * H, transcendentals=0,
          bytes_accessed=2 * ne * H * 4),
  )


def _make_scatter_x(ne):
  epw = ne // NW
  ngrp = epw // L
  rem = epw - ngrp * L
  epw_pad = epw + (L - rem if rem else 0)

  def body(src_ref, dst_ref, w_hbm, xq_hbm, zq_ref,
           xcpart_ref,
           idxs_v, idxd_v, w_v, xq_v, acc_xc):
    cid = lax.axis_index("c")
    sid = lax.axis_index("s")
    wid = sid * NC + cid
    base = wid * epw

    pltpu.sync_copy(src_ref.at[pl.ds(base, epw)], idxs_v.at[pl.ds(0, epw)])
    pltpu.sync_copy(dst_ref.at[pl.ds(base, epw)], idxd_v.at[pl.ds(0, epw)])
    pltpu.sync_copy(w_hbm.at[pl.ds(base, epw)], w_v.at[pl.ds(0, epw)])
    pltpu.sync_copy(xq_hbm, xq_v)
    pltpu.sync_copy(zq_ref, acc_xc)

    ones = jnp.full((L,), 1.0, _f32)

    def group(g, carry):
      sl = pl.ds(g * L, L)
      isrc = idxs_v[sl] * XC
      idst = idxd_v[sl] * XC
      wv = w_v[sl]
      for c in range(3):
        xs = plsc.load_gather(xq_v, [isrc + c])
        xd = plsc.load_gather(xq_v, [idst + c])
        plsc.addupdate_scatter(acc_xc, [idst + c], wv * (xs - xd))
      plsc.addupdate_scatter(acc_xc, [idst + 3], ones)
      return carry

    lax.fori_loop(0, ngrp, group, 0)

    if rem:
      sl = pl.ds(ngrp * L, L)
      mask = lax.iota(jnp.int32, L) < rem
      isrc = jnp.where(mask, idxs_v[sl], 0) * XC
      idst = jnp.where(mask, idxd_v[sl], 0) * XC
      wv = jnp.where(mask, w_v[sl], 0.0)
      for c in range(3):
        xs = plsc.load_gather(xq_v, [isrc + c])
        xd = plsc.load_gather(xq_v, [idst + c])
        plsc.addupdate_scatter(acc_xc, [idst + c], wv * (xs - xd))
      plsc.addupdate_scatter(acc_xc, [idst + 3],
                             jnp.where(mask, 1.0, 0.0).astype(_f32))

    pltpu.sync_copy(acc_xc, xcpart_ref.at[pl.ds(wid * (N * XC), N * XC)])

  return pl.kernel(
      body,
      out_type=jax.ShapeDtypeStruct((NW * N * XC,), _f32),
      mesh=_mesh,
      scratch_types=[
          pltpu.VMEM((epw_pad,), jnp.int32),
          pltpu.VMEM((epw_pad,), jnp.int32),
          pltpu.VMEM((epw_pad,), _f32),
          pltpu.VMEM((N * XC,), _f32),
          pltpu.VMEM((N * XC,), _f32),
      ],
      compiler_params=pltpu.CompilerParams(needs_layout_passes=False),
      cost_estimate=pl.CostEstimate(
          flops=8 * ne, transcendentals=0,
          bytes_accessed=3 * ne * 4 + NW * N * XC * 4),
  )


_gather_c = _make_gather(EC)
_edge_c = _make_edge(EC)
_scatter_h_c = _make_scatter_h(EC)
_scatter_x_c = _make_scatter_x(EC)


def _xred_body(xcp_ref, xsum_ref):
  xsum_ref[:] = jnp.sum(xcp_ref[:], axis=0)


_xred = pl.pallas_call(
    _xred_body,
    out_shape=jax.ShapeDtypeStruct((N * XC,), _f32),
)


def _fold_body(h_ref, x4_ref, hp_ref, xs_ref, wn2_ref, bn2_ref,
               hout_ref, xout_ref):
  hsum = hp_ref[0]
  for p in range(1, NC * CH):
    hsum = hsum + hp_ref[p]
  xsum = xs_ref[:]
  deg = xsum[:, 3:4]
  hout_ref[:] = (h_ref[:]
                 + jnp.dot(hsum, wn2_ref[:], preferred_element_type=_f32)
                 + deg * bn2_ref[:])
  xout_ref[:] = x4_ref[:] + xsum


_fold = pl.pallas_call(
    _fold_body,
    out_shape=[jax.ShapeDtypeStruct((N, H), _f32),
               jax.ShapeDtypeStruct((N, XC), _f32)],
)


def kernel(h, x, edge_index, edge_dist,
           W_e1, b_e1, W_e2, b_e2,
           W_n1, b_n1, W_n2, b_n2,
           W_c1, b_c1, W_c2):
  src = edge_index[0].astype(jnp.int32)
  dst = edge_index[1].astype(jnp.int32)
  x4 = jnp.concatenate([x.astype(_f32), jnp.zeros((N, 1), _f32)], axis=1)
  xq = x4.reshape(N * XC)

  wsrc = jnp.concatenate([W_n1[:D], W_c1[:D]], axis=1)
  wdst = jnp.concatenate([W_n1[D:2 * D], W_c1[D:2 * D]], axis=1)
  vcat = jnp.concatenate([W_n1[2 * D:], W_c1[2 * D:]], axis=1)
  bcat = jnp.concatenate([b_n1, b_c1]).reshape(1, PW)
  be1r = b_e1.reshape(1, ED)
  be2r = b_e2.reshape(1, ED)
  wc2r = W_c2.reshape(1, H)

  psrc, pdst = _proj(h, wsrc, wdst)
  zh = jnp.zeros((N, H), _f32)
  zq = jnp.zeros((N * XC,), _f32)

  srcs = [lax.slice(src, (c * EC,), ((c + 1) * EC,)) for c in range(CH)]
  dsts = [lax.slice(dst, (c * EC,), ((c + 1) * EC,)) for c in range(CH)]
  dists = [lax.slice(edge_dist, (c * EC,), ((c + 1) * EC,))
           for c in range(CH)]

  z0s = [_gather_c(srcs[c], dsts[c], psrc, pdst) for c in range(CH)]
  svw = [_edge_c(dists[c], z0s[c], W_e1, be1r, W_e2, be2r, vcat, bcat, wc2r)
         for c in range(CH)]
  hparts = [_scatter_h_c(dsts[c], svw[c][0], zh) for c in range(CH)]
  xcs = [_scatter_x_c(srcs[c], dsts[c], svw[c][1].reshape(EC), xq, zq)
         for c in range(CH)]

  xsum = _xred(jnp.concatenate(
      [xc.reshape(NW, N * XC) for xc in xcs], axis=0))
  hout, xout4 = _fold(h, x4, jnp.concatenate(hparts, axis=0),
                      xsum.reshape(N, XC), W_n2, b_n2.reshape(1, H))
  return hout, xout4[:, :3]


# speedup vs baseline: 1.0860x; 1.0860x over previous
import jax
import jax.numpy as jnp
from jax import lax
from jax.experimental import pallas as pl
from jax.experimental.pallas import tpu as pltpu
from jax.experimental.pallas import tpu_sc as plsc

N = 10000
E = 320000
D = 128
H = 128
ED = 16
XC = 4
PW = 2 * H

NC = 2
NS = 16
NW = NC * NS
L = 16
WIN = 80

CH = 1
EC = E // CH

ROWS_A = 632
ROWS_B = N - (NS - 1) * ROWS_A

_mesh = plsc.VectorSubcoreMesh(
    core_axis_name="c", subcore_axis_name="s", num_cores=NC, num_subcores=NS)

_f32 = jnp.float32


def _sig(t):
  return 1.0 / (1.0 + jnp.exp(-t))


def _proj_body(h_ref, wsrc_ref, wdst_ref, psrc_ref, pdst_ref):
  hb = h_ref[:]
  psrc_ref[:] = jnp.dot(hb, wsrc_ref[:], preferred_element_type=_f32)
  pdst_ref[:] = jnp.dot(hb, wdst_ref[:], preferred_element_type=_f32)


_proj = pl.pallas_call(
    _proj_body,
    out_shape=[jax.ShapeDtypeStruct((N, PW), _f32),
               jax.ShapeDtypeStruct((N, PW), _f32)],
)


def _make_gather(ne):
  epw = ne // NW
  nwin = epw // WIN

  def body(src_ref, dst_ref, psrc_ref, pdst_ref,
           z0_ref,
           idxs_v, idxd_v, gs0, gd0, gs1, gd1,
           sem_a0, sem_b0, sem_a1, sem_b1):
    wid = lax.axis_index("s") * NC + lax.axis_index("c")
    base = wid * epw

    pltpu.sync_copy(src_ref.at[pl.ds(base, epw)], idxs_v)
    pltpu.sync_copy(dst_ref.at[pl.ds(base, epw)], idxd_v)

    def issue(w, gs, gd, sa, sb):
      sl = pl.ds(w * WIN, WIN)
      pltpu.async_copy(psrc_ref.at[idxs_v.at[sl]], gs, sa)
      pltpu.async_copy(pdst_ref.at[idxd_v.at[sl]], gd, sb)

    def process(w, gs, gd, sa, sb):
      pltpu.make_async_copy(
          psrc_ref.at[idxs_v.at[pl.ds(0, WIN)]], gs, sa).wait()
      pltpu.make_async_copy(
          pdst_ref.at[idxd_v.at[pl.ds(0, WIN)]], gd, sb).wait()

      def row(i, c2):
        for k in range(PW // L):
          sl = pl.ds(k * L, L)
          gs[i, sl] = gs[i, sl] + gd[i, sl]
        return c2

      lax.fori_loop(0, WIN, row, 0)
      pltpu.sync_copy(gs, z0_ref.at[pl.ds(base + w * WIN, WIN)])

    issue(0, gs0, gd0, sem_a0, sem_b0)

    def pair(k, carry):
      w0 = 2 * k
      issue(w0 + 1, gs1, gd1, sem_a1, sem_b1)
      process(w0, gs0, gd0, sem_a0, sem_b0)

      @pl.when(w0 + 2 < nwin)
      def _():
        issue(w0 + 2, gs0, gd0, sem_a0, sem_b0)

      process(w0 + 1, gs1, gd1, sem_a1, sem_b1)
      return carry

    lax.fori_loop(0, nwin // 2, pair, 0)
    if nwin % 2 == 1:
      process(nwin - 1, gs0, gd0, sem_a0, sem_b0)

  return pl.kernel(
      body,
      out_type=jax.ShapeDtypeStruct((ne, PW), _f32),
      mesh=_mesh,
      scratch_types=[
          pltpu.VMEM((epw,), jnp.int32),
          pltpu.VMEM((epw,), jnp.int32),
          pltpu.VMEM((WIN, PW), _f32),
          pltpu.VMEM((WIN, PW), _f32),
          pltpu.VMEM((WIN, PW), _f32),
          pltpu.VMEM((WIN, PW), _f32),
          pltpu.SemaphoreType.DMA,
          pltpu.SemaphoreType.DMA,
          pltpu.SemaphoreType.DMA,
          pltpu.SemaphoreType.DMA,
      ],
      cost_estimate=pl.CostEstimate(
          flops=ne * PW, transcendentals=0,
          bytes_accessed=3 * ne * PW * 4),
  )


BE = 3200


def _make_edge(ne):
  def body(dist_ref, z0_ref,
           we1_ref, be1_ref, we2_ref, be2_ref,
           vcat_ref, bcat_ref, wc2_ref,
           sv_ref, w_ref):
    i = pl.program_id(0)
    d = dist_ref[pl.ds(i * BE, BE)]
    e1 = d[:, None] * we1_ref[:] + be1_ref[:]
    e1 = e1 * _sig(e1)
    e2 = jnp.dot(e1, we2_ref[:], preferred_element_type=_f32) + be2_ref[:]
    z = z0_ref[:] + jnp.dot(e2, vcat_ref[:], preferred_element_type=_f32)
    z = z + bcat_ref[:]
    s = z * _sig(z)
    sv_ref[:] = s[:, :H]
    s_c3 = s[:, H:].reshape(BE // 128, 128, H)
    wc2b = jnp.broadcast_to(wc2_ref[:].reshape(1, 1, H), (BE // 128, 1, H))
    wt = lax.dot_general(wc2b, s_c3, (((2,), (2,)), ((0,), (0,))),
                         preferred_element_type=_f32)
    w_ref[:] = wt.reshape(1, BE // 128, 128)

  return pl.pallas_call(
      body,
      grid=(ne // BE,),
      in_specs=[
          pl.BlockSpec((ne,), lambda i: (0,)),
          pl.BlockSpec((BE, PW), lambda i: (i, 0)),
          pl.BlockSpec((1, ED), lambda i: (0, 0)),
          pl.BlockSpec((1, ED), lambda i: (0, 0)),
          pl.BlockSpec((ED, ED), lambda i: (0, 0)),
          pl.BlockSpec((1, ED), lambda i: (0, 0)),
          pl.BlockSpec((ED, PW), lambda i: (0, 0)),
          pl.BlockSpec((1, PW), lambda i: (0, 0)),
          pl.BlockSpec((1, H), lambda i: (0, 0)),
      ],
      out_specs=[
          pl.BlockSpec((BE, H), lambda i: (i, 0)),
          pl.BlockSpec((1, BE // 128, 128), lambda i: (i, 0, 0)),
      ],
      out_shape=[jax.ShapeDtypeStruct((ne, H), _f32),
                 jax.ShapeDtypeStruct((ne // BE, BE // 128, 128), _f32)],
      compiler_params=pltpu.CompilerParams(
          dimension_semantics=("arbitrary",)),
  )


def _make_scatter_h(ne):
  epw = ne // NW
  nwin = epw // WIN

  def body(dst_ref, sv_hbm, zh_ref, hpart_ref,
           idxd0, idxd1, sv0, sv1, acc_s,
           sem_i0, sem_s0, sem_i1, sem_s1):
    cid = lax.axis_index("c")
    sid = lax.axis_index("s")
    wid = sid * NC + cid
    base = wid * epw
    row0 = sid * ROWS_A

    @pl.when(sid < NS - 1)
    def _():
      pltpu.sync_copy(zh_ref.at[pl.ds(row0, ROWS_A)],
                      acc_s.at[pl.ds(row0, ROWS_A)])

    @pl.when(sid == NS - 1)
    def _():
      pltpu.sync_copy(zh_ref.at[pl.ds((NS - 1) * ROWS_A, ROWS_B)],
                      acc_s.at[pl.ds((NS - 1) * ROWS_A, ROWS_B)])

    plsc.subcore_barrier()

    def issue(w, idx_v, sv_v, si, ss):
      off = base + w * WIN
      pltpu.async_copy(dst_ref.at[pl.ds(off, WIN)], idx_v, si)
      pltpu.async_copy(sv_hbm.at[pl.ds(off, WIN)], sv_v, ss)

    def process(idx_v, sv_v, si, ss):
      pltpu.make_async_copy(dst_ref.at[pl.ds(0, WIN)], idx_v, si).wait()
      pltpu.make_async_copy(sv_hbm.at[pl.ds(0, WIN)], sv_v, ss).wait()
      pltpu.sync_copy(sv_v, acc_s.at[idx_v], add=True)

    issue(0, idxd0, sv0, sem_i0, sem_s0)

    def pair(k, carry):
      issue(2 * k + 1, idxd1, sv1, sem_i1, sem_s1)
      process(idxd0, sv0, sem_i0, sem_s0)

      @pl.when(2 * k + 2 < nwin)
      def _():
        issue(2 * k + 2, idxd0, sv0, sem_i0, sem_s0)

      process(idxd1, sv1, sem_i1, sem_s1)
      return carry

    lax.fori_loop(0, nwin // 2, pair, 0)
    if nwin % 2 == 1:
      process(idxd0, sv0, sem_i0, sem_s0)

    plsc.subcore_barrier()

    @pl.when(sid < NS - 1)
    def _():
      pltpu.sync_copy(acc_s.at[pl.ds(row0, ROWS_A)],
                      hpart_ref.at[cid, pl.ds(row0, ROWS_A)])

    @pl.when(sid == NS - 1)
    def _():
      pltpu.sync_copy(acc_s.at[pl.ds((NS - 1) * ROWS_A, ROWS_B)],
                      hpart_ref.at[cid, pl.ds((NS - 1) * ROWS_A, ROWS_B)])

  return pl.kernel(
      body,
      out_type=jax.ShapeDtypeStruct((NC, N, H), _f32),
      mesh=_mesh,
      scratch_types=[
          pltpu.VMEM((WIN,), jnp.int32),
          pltpu.VMEM((WIN,), jnp.int32),
          pltpu.VMEM((WIN, H), _f32),
          pltpu.VMEM((WIN, H), _f32),
          pltpu.VMEM_SHARED((N, H), _f32),
          pltpu.SemaphoreType.DMA,
          pltpu.SemaphoreType.DMA,
          pltpu.SemaphoreType.DMA,
          pltpu.SemaphoreType.DMA,
      ],
      cost_estimate=pl.CostEstimate(
          flops=ne * H, transcendentals=0,
          bytes_accessed=2 * ne * H * 4),
  )


def _make_scatter_x(ne):
  epw = ne // NW
  ngrp = epw // L
  rem = epw - ngrp * L
  epw_pad = epw + (L - rem if rem else 0)

  def body(src_ref, dst_ref, w_hbm, xq_hbm, zq_ref,
           xcpart_ref,
           idxs_v, idxd_v, w_v, xq_v, acc_xc):
    cid = lax.axis_index("c")
    sid = lax.axis_index("s")
    wid = sid * NC + cid
    base = wid * epw

    pltpu.sync_copy(src_ref.at[pl.ds(base, epw)], idxs_v.at[pl.ds(0, epw)])
    pltpu.sync_copy(dst_ref.at[pl.ds(base, epw)], idxd_v.at[pl.ds(0, epw)])
    pltpu.sync_copy(w_hbm.at[pl.ds(base, epw)], w_v.at[pl.ds(0, epw)])
    pltpu.sync_copy(xq_hbm, xq_v)
    pltpu.sync_copy(zq_ref, acc_xc)

    ones = jnp.full((L,), 1.0, _f32)

    def group(g, carry):
      sl = pl.ds(g * L, L)
      isrc = idxs_v[sl] * XC
      idst = idxd_v[sl] * XC
      wv = w_v[sl]
      for c in range(3):
        xs = plsc.load_gather(xq_v, [isrc + c])
        xd = plsc.load_gather(xq_v, [idst + c])
        plsc.addupdate_scatter(acc_xc, [idst + c], wv * (xs - xd))
      plsc.addupdate_scatter(acc_xc, [idst + 3], ones)
      return carry

    lax.fori_loop(0, ngrp, group, 0)

    if rem:
      sl = pl.ds(ngrp * L, L)
      mask = lax.iota(jnp.int32, L) < rem
      isrc = jnp.where(mask, idxs_v[sl], 0) * XC
      idst = jnp.where(mask, idxd_v[sl], 0) * XC
      wv = jnp.where(mask, w_v[sl], 0.0)
      for c in range(3):
        xs = plsc.load_gather(xq_v, [isrc + c])
        xd = plsc.load_gather(xq_v, [idst + c])
        plsc.addupdate_scatter(acc_xc, [idst + c], wv * (xs - xd))
      plsc.addupdate_scatter(acc_xc, [idst + 3],
                             jnp.where(mask, 1.0, 0.0).astype(_f32))

    pltpu.sync_copy(acc_xc, xcpart_ref.at[pl.ds(wid * (N * XC), N * XC)])

  return pl.kernel(
      body,
      out_type=jax.ShapeDtypeStruct((NW * N * XC,), _f32),
      mesh=_mesh,
      scratch_types=[
          pltpu.VMEM((epw_pad,), jnp.int32),
          pltpu.VMEM((epw_pad,), jnp.int32),
          pltpu.VMEM((epw_pad,), _f32),
          pltpu.VMEM((N * XC,), _f32),
          pltpu.VMEM((N * XC,), _f32),
      ],
      compiler_params=pltpu.CompilerParams(needs_layout_passes=False),
      cost_estimate=pl.CostEstimate(
          flops=8 * ne, transcendentals=0,
          bytes_accessed=3 * ne * 4 + NW * N * XC * 4),
  )


_gather_c = _make_gather(EC)
_edge_c = _make_edge(EC)
_scatter_h_c = _make_scatter_h(EC)
_scatter_x_c = _make_scatter_x(EC)


def _xred_body(xcp_ref, xsum_ref):
  xsum_ref[:] = jnp.sum(xcp_ref[:], axis=0)


_xred = pl.pallas_call(
    _xred_body,
    out_shape=jax.ShapeDtypeStruct((N * XC,), _f32),
)


def _fold_body(h_ref, x4_ref, hp_ref, xs_ref, wn2_ref, bn2_ref,
               hout_ref, xout_ref):
  hsum = hp_ref[0]
  for p in range(1, NC * CH):
    hsum = hsum + hp_ref[p]
  xsum = xs_ref[:]
  deg = xsum[:, 3:4]
  hout_ref[:] = (h_ref[:]
                 + jnp.dot(hsum, wn2_ref[:], preferred_element_type=_f32)
                 + deg * bn2_ref[:])
  xout_ref[:] = x4_ref[:] + xsum


_fold = pl.pallas_call(
    _fold_body,
    out_shape=[jax.ShapeDtypeStruct((N, H), _f32),
               jax.ShapeDtypeStruct((N, XC), _f32)],
)


def kernel(h, x, edge_index, edge_dist,
           W_e1, b_e1, W_e2, b_e2,
           W_n1, b_n1, W_n2, b_n2,
           W_c1, b_c1, W_c2):
  src = edge_index[0].astype(jnp.int32)
  dst = edge_index[1].astype(jnp.int32)
  x4 = jnp.concatenate([x.astype(_f32), jnp.zeros((N, 1), _f32)], axis=1)
  xq = x4.reshape(N * XC)

  wsrc = jnp.concatenate([W_n1[:D], W_c1[:D]], axis=1)
  wdst = jnp.concatenate([W_n1[D:2 * D], W_c1[D:2 * D]], axis=1)
  vcat = jnp.concatenate([W_n1[2 * D:], W_c1[2 * D:]], axis=1)
  bcat = jnp.concatenate([b_n1, b_c1]).reshape(1, PW)
  be1r = b_e1.reshape(1, ED)
  be2r = b_e2.reshape(1, ED)
  wc2r = W_c2.reshape(1, H)

  psrc, pdst = _proj(h, wsrc, wdst)
  zh = jnp.zeros((N, H), _f32)
  zq = jnp.zeros((N * XC,), _f32)

  srcs = [lax.slice(src, (c * EC,), ((c + 1) * EC,)) for c in range(CH)]
  dsts = [lax.slice(dst, (c * EC,), ((c + 1) * EC,)) for c in range(CH)]
  dists = [lax.slice(edge_dist, (c * EC,), ((c + 1) * EC,))
           for c in range(CH)]

  z0s = [_gather_c(srcs[c], dsts[c], psrc, pdst) for c in range(CH)]
  svw = [_edge_c(dists[c], z0s[c], W_e1, be1r, W_e2, be2r, vcat, bcat, wc2r)
         for c in range(CH)]
  hparts = [_scatter_h_c(dsts[c], svw[c][0], zh) for c in range(CH)]
  xcs = [_scatter_x_c(srcs[c], dsts[c], svw[c][1].reshape(EC), xq, zq)
         for c in range(CH)]

  xsum = _xred(jnp.concatenate(
      [xc.reshape(NW, N * XC) for xc in xcs], axis=0))
  hout, xout4 = _fold(h, x4, jnp.concatenate(hparts, axis=0),
                      xsum.reshape(N, XC), W_n2, b_n2.reshape(1, H))
  return hout, xout4[:, :3]


# speedup vs baseline: 1.2818x; 1.1803x over previous
import jax
import jax.numpy as jnp
from jax import lax
from jax.experimental import pallas as pl
from jax.experimental.pallas import tpu as pltpu
from jax.experimental.pallas import tpu_sc as plsc

N = 10000
E = 320000
D = 128
H = 128
ED = 16
XC = 4
PW = 2 * H

NC = 2
NS = 16
NW = NC * NS
L = 16
WIN = 80

CH = 1
EC = E // CH

ROWS_A = 632
ROWS_B = N - (NS - 1) * ROWS_A

_mesh = plsc.VectorSubcoreMesh(
    core_axis_name="c", subcore_axis_name="s", num_cores=NC, num_subcores=NS)

_f32 = jnp.float32
_bf16 = jnp.bfloat16


def _sig(t):
  return 1.0 / (1.0 + jnp.exp(-t))


def _rne_bf16_bits(f):
  u = lax.bitcast_convert_type(f, jnp.int32)
  u = u + 0x7FFF + (lax.shift_right_logical(u, 16) & 1)
  return lax.shift_right_logical(u, 16)


def _proj_body(h_ref, wsrc_ref, wdst_ref, psrc_ref, pdst_ref):
  hb = h_ref[:]
  for wref, pref in ((wsrc_ref, psrc_ref), (wdst_ref, pdst_ref)):
    pn = jnp.dot(hb, wref[:, :H], preferred_element_type=_f32)
    pc = jnp.dot(hb, wref[:, H:], preferred_element_type=_f32)
    pref[:] = _rne_bf16_bits(pn) | lax.shift_left(_rne_bf16_bits(pc), 16)


_proj = pl.pallas_call(
    _proj_body,
    out_shape=[jax.ShapeDtypeStruct((N, H), jnp.int32),
               jax.ShapeDtypeStruct((N, H), jnp.int32)],
)


def _make_gather(ne):
  epw = ne // NW
  nwin = epw // WIN

  def body(src_ref, dst_ref, psrc_ref, pdst_ref,
           za_ref, zb_ref,
           idxs_v, idxd_v, gs0, gd0, gs1, gd1,
           sem_a0, sem_b0, sem_a1, sem_b1):
    wid = lax.axis_index("s") * NC + lax.axis_index("c")
    base = wid * epw

    pltpu.sync_copy(src_ref.at[pl.ds(base, epw)], idxs_v)
    pltpu.sync_copy(dst_ref.at[pl.ds(base, epw)], idxd_v)

    def issue(w, gs, gd, sa, sb):
      sl = pl.ds(w * WIN, WIN)
      pltpu.async_copy(psrc_ref.at[idxs_v.at[sl]], gs, sa)
      pltpu.async_copy(pdst_ref.at[idxd_v.at[sl]], gd, sb)

    def process(w, gs, gd, sa, sb):
      pltpu.make_async_copy(
          psrc_ref.at[idxs_v.at[pl.ds(0, WIN)]], gs, sa).wait()
      pltpu.make_async_copy(
          pdst_ref.at[idxd_v.at[pl.ds(0, WIN)]], gd, sb).wait()
      pltpu.sync_copy(gs, za_ref.at[pl.ds(base + w * WIN, WIN)])
      pltpu.sync_copy(gd, zb_ref.at[pl.ds(base + w * WIN, WIN)])

    issue(0, gs0, gd0, sem_a0, sem_b0)

    def pair(k, carry):
      w0 = 2 * k
      issue(w0 + 1, gs1, gd1, sem_a1, sem_b1)
      process(w0, gs0, gd0, sem_a0, sem_b0)

      @pl.when(w0 + 2 < nwin)
      def _():
        issue(w0 + 2, gs0, gd0, sem_a0, sem_b0)

      process(w0 + 1, gs1, gd1, sem_a1, sem_b1)
      return carry

    lax.fori_loop(0, nwin // 2, pair, 0)
    if nwin % 2 == 1:
      process(nwin - 1, gs0, gd0, sem_a0, sem_b0)

  return pl.kernel(
      body,
      out_type=[jax.ShapeDtypeStruct((ne, H), jnp.int32),
                jax.ShapeDtypeStruct((ne, H), jnp.int32)],
      mesh=_mesh,
      scratch_types=[
          pltpu.VMEM((epw,), jnp.int32),
          pltpu.VMEM((epw,), jnp.int32),
          pltpu.VMEM((WIN, H), jnp.int32),
          pltpu.VMEM((WIN, H), jnp.int32),
          pltpu.VMEM((WIN, H), jnp.int32),
          pltpu.VMEM((WIN, H), jnp.int32),
          pltpu.SemaphoreType.DMA,
          pltpu.SemaphoreType.DMA,
          pltpu.SemaphoreType.DMA,
          pltpu.SemaphoreType.DMA,
      ],
      cost_estimate=pl.CostEstimate(
          flops=ne * PW, transcendentals=0,
          bytes_accessed=4 * ne * H * 4),
  )


BE = 3200


def _unpack_lo(zi):
  return lax.bitcast_convert_type(lax.shift_left(zi, 16), _f32)


def _unpack_hi(zi):
  return lax.bitcast_convert_type(zi & jnp.int32(-65536), _f32)


def _make_edge(ne):
  def body(dist_ref, za_ref, zb_ref,
           we1_ref, be1_ref, we2_ref, be2_ref,
           vn_ref, vc_ref, bn_ref, bc_ref, wc2_ref,
           sv_ref, w_ref):
    i = pl.program_id(0)
    d = dist_ref[pl.ds(i * BE, BE)]
    e1 = d[:, None] * we1_ref[:] + be1_ref[:]
    e1 = e1 * _sig(e1)
    e2 = jnp.dot(e1, we2_ref[:], preferred_element_type=_f32) + be2_ref[:]
    za = za_ref[:]
    zb = zb_ref[:]
    zn = (_unpack_lo(za) + _unpack_lo(zb)
          + jnp.dot(e2, vn_ref[:], preferred_element_type=_f32) + bn_ref[:])
    zc = (_unpack_hi(za) + _unpack_hi(zb)
          + jnp.dot(e2, vc_ref[:], preferred_element_type=_f32) + bc_ref[:])
    sv_ref[:] = zn * _sig(zn)
    s_c = zc * _sig(zc)
    s_c3 = s_c.reshape(BE // 128, 128, H)
    wc2b = jnp.broadcast_to(wc2_ref[:].reshape(1, 1, H), (BE // 128, 1, H))
    wt = lax.dot_general(wc2b, s_c3, (((2,), (2,)), ((0,), (0,))),
                         preferred_element_type=_f32)
    w_ref[:] = wt.reshape(1, BE // 128, 128)

  return pl.pallas_call(
      body,
      grid=(ne // BE,),
      in_specs=[
          pl.BlockSpec((ne,), lambda i: (0,)),
          pl.BlockSpec((BE, H), lambda i: (i, 0)),
          pl.BlockSpec((BE, H), lambda i: (i, 0)),
          pl.BlockSpec((1, ED), lambda i: (0, 0)),
          pl.BlockSpec((1, ED), lambda i: (0, 0)),
          pl.BlockSpec((ED, ED), lambda i: (0, 0)),
          pl.BlockSpec((1, ED), lambda i: (0, 0)),
          pl.BlockSpec((ED, H), lambda i: (0, 0)),
          pl.BlockSpec((ED, H), lambda i: (0, 0)),
          pl.BlockSpec((1, H), lambda i: (0, 0)),
          pl.BlockSpec((1, H), lambda i: (0, 0)),
          pl.BlockSpec((1, H), lambda i: (0, 0)),
      ],
      out_specs=[
          pl.BlockSpec((BE, H), lambda i: (i, 0)),
          pl.BlockSpec((1, BE // 128, 128), lambda i: (i, 0, 0)),
      ],
      out_shape=[jax.ShapeDtypeStruct((ne, H), _f32),
                 jax.ShapeDtypeStruct((ne // BE, BE // 128, 128), _f32)],
      compiler_params=pltpu.CompilerParams(
          dimension_semantics=("arbitrary",)),
  )


def _make_scatter_h(ne):
  epw = ne // NW
  nwin = epw // WIN

  def body(dst_ref, sv_hbm, zh_ref, hpart_ref,
           idxd0, idxd1, sv0, sv1, acc_s,
           sem_i0, sem_s0, sem_i1, sem_s1):
    cid = lax.axis_index("c")
    sid = lax.axis_index("s")
    wid = sid * NC + cid
    base = wid * epw
    row0 = sid * ROWS_A

    @pl.when(sid < NS - 1)
    def _():
      pltpu.sync_copy(zh_ref.at[pl.ds(row0, ROWS_A)],
                      acc_s.at[pl.ds(row0, ROWS_A)])

    @pl.when(sid == NS - 1)
    def _():
      pltpu.sync_copy(zh_ref.at[pl.ds((NS - 1) * ROWS_A, ROWS_B)],
                      acc_s.at[pl.ds((NS - 1) * ROWS_A, ROWS_B)])

    plsc.subcore_barrier()

    def issue(w, idx_v, sv_v, si, ss):
      off = base + w * WIN
      pltpu.async_copy(dst_ref.at[pl.ds(off, WIN)], idx_v, si)
      pltpu.async_copy(sv_hbm.at[pl.ds(off, WIN)], sv_v, ss)

    def process(idx_v, sv_v, si, ss):
      pltpu.make_async_copy(dst_ref.at[pl.ds(0, WIN)], idx_v, si).wait()
      pltpu.make_async_copy(sv_hbm.at[pl.ds(0, WIN)], sv_v, ss).wait()
      pltpu.sync_copy(sv_v, acc_s.at[idx_v], add=True)

    issue(0, idxd0, sv0, sem_i0, sem_s0)

    def pair(k, carry):
      issue(2 * k + 1, idxd1, sv1, sem_i1, sem_s1)
      process(idxd0, sv0, sem_i0, sem_s0)

      @pl.when(2 * k + 2 < nwin)
      def _():
        issue(2 * k + 2, idxd0, sv0, sem_i0, sem_s0)

      process(idxd1, sv1, sem_i1, sem_s1)
      return carry

    lax.fori_loop(0, nwin // 2, pair, 0)
    if nwin % 2 == 1:
      process(idxd0, sv0, sem_i0, sem_s0)

    plsc.subcore_barrier()

    @pl.when(sid < NS - 1)
    def _():
      pltpu.sync_copy(acc_s.at[pl.ds(row0, ROWS_A)],
                      hpart_ref.at[cid, pl.ds(row0, ROWS_A)])

    @pl.when(sid == NS - 1)
    def _():
      pltpu.sync_copy(acc_s.at[pl.ds((NS - 1) * ROWS_A, ROWS_B)],
                      hpart_ref.at[cid, pl.ds((NS - 1) * ROWS_A, ROWS_B)])

  return pl.kernel(
      body,
      out_type=jax.ShapeDtypeStruct((NC, N, H), _f32),
      mesh=_mesh,
      scratch_types=[
          pltpu.VMEM((WIN,), jnp.int32),
          pltpu.VMEM((WIN,), jnp.int32),
          pltpu.VMEM((WIN, H), _f32),
          pltpu.VMEM((WIN, H), _f32),
          pltpu.VMEM_SHARED((N, H), _f32),
          pltpu.SemaphoreType.DMA,
          pltpu.SemaphoreType.DMA,
          pltpu.SemaphoreType.DMA,
          pltpu.SemaphoreType.DMA,
      ],
      cost_estimate=pl.CostEstimate(
          flops=ne * H, transcendentals=0,
          bytes_accessed=2 * ne * H * 4),
  )


def _make_scatter_x(ne):
  epw = ne // NW
  ngrp = epw // L
  rem = epw - ngrp * L
  epw_pad = epw + (L - rem if rem else 0)

  def body(src_ref, dst_ref, w_hbm, xq_hbm, zq_ref,
           xcpart_ref,
           idxs_v, idxd_v, w_v, xq_v, acc_xc):
    cid = lax.axis_index("c")
    sid = lax.axis_index("s")
    wid = sid * NC + cid
    base = wid * epw

    pltpu.sync_copy(src_ref.at[pl.ds(base, epw)], idxs_v.at[pl.ds(0, epw)])
    pltpu.sync_copy(dst_ref.at[pl.ds(base, epw)], idxd_v.at[pl.ds(0, epw)])
    pltpu.sync_copy(w_hbm.at[pl.ds(base, epw)], w_v.at[pl.ds(0, epw)])
    pltpu.sync_copy(xq_hbm, xq_v)
    pltpu.sync_copy(zq_ref, acc_xc)

    ones = jnp.full((L,), 1.0, _f32)

    def group(g, carry):
      sl = pl.ds(g * L, L)
      isrc = idxs_v[sl] * XC
      idst = idxd_v[sl] * XC
      wv = w_v[sl]
      for c in range(3):
        xs = plsc.load_gather(xq_v, [isrc + c])
        xd = plsc.load_gather(xq_v, [idst + c])
        plsc.addupdate_scatter(acc_xc, [idst + c], wv * (xs - xd))
      plsc.addupdate_scatter(acc_xc, [idst + 3], ones)
      return carry

    lax.fori_loop(0, ngrp, group, 0)

    if rem:
      sl = pl.ds(ngrp * L, L)
      mask = lax.iota(jnp.int32, L) < rem
      isrc = jnp.where(mask, idxs_v[sl], 0) * XC
      idst = jnp.where(mask, idxd_v[sl], 0) * XC
      wv = jnp.where(mask, w_v[sl], 0.0)
      for c in range(3):
        xs = plsc.load_gather(xq_v, [isrc + c])
        xd = plsc.load_gather(xq_v, [idst + c])
        plsc.addupdate_scatter(acc_xc, [idst + c], wv * (xs - xd))
      plsc.addupdate_scatter(acc_xc, [idst + 3],
                             jnp.where(mask, 1.0, 0.0).astype(_f32))

    pltpu.sync_copy(acc_xc, xcpart_ref.at[pl.ds(wid * (N * XC), N * XC)])

  return pl.kernel(
      body,
      out_type=jax.ShapeDtypeStruct((NW * N * XC,), _f32),
      mesh=_mesh,
      scratch_types=[
          pltpu.VMEM((epw_pad,), jnp.int32),
          pltpu.VMEM((epw_pad,), jnp.int32),
          pltpu.VMEM((epw_pad,), _f32),
          pltpu.VMEM((N * XC,), _f32),
          pltpu.VMEM((N * XC,), _f32),
      ],
      compiler_params=pltpu.CompilerParams(needs_layout_passes=False),
      cost_estimate=pl.CostEstimate(
          flops=8 * ne, transcendentals=0,
          bytes_accessed=3 * ne * 4 + NW * N * XC * 4),
  )


_gather_c = _make_gather(EC)
_edge_c = _make_edge(EC)
_scatter_h_c = _make_scatter_h(EC)
_scatter_x_c = _make_scatter_x(EC)


def _xred_body(xcp_ref, xsum_ref):
  xsum_ref[:] = jnp.sum(xcp_ref[:], axis=0)


_xred = pl.pallas_call(
    _xred_body,
    out_shape=jax.ShapeDtypeStruct((N * XC,), _f32),
)


def _fold_body(h_ref, x4_ref, hp_ref, xs_ref, wn2_ref, bn2_ref,
               hout_ref, xout_ref):
  hsum = hp_ref[0]
  for p in range(1, NC * CH):
    hsum = hsum + hp_ref[p]
  xsum = xs_ref[:]
  deg = xsum[:, 3:4]
  hout_ref[:] = (h_ref[:]
                 + jnp.dot(hsum, wn2_ref[:], preferred_element_type=_f32)
                 + deg * bn2_ref[:])
  xout_ref[:] = x4_ref[:] + xsum


_fold = pl.pallas_call(
    _fold_body,
    out_shape=[jax.ShapeDtypeStruct((N, H), _f32),
               jax.ShapeDtypeStruct((N, XC), _f32)],
)


def kernel(h, x, edge_index, edge_dist,
           W_e1, b_e1, W_e2, b_e2,
           W_n1, b_n1, W_n2, b_n2,
           W_c1, b_c1, W_c2):
  src = edge_index[0].astype(jnp.int32)
  dst = edge_index[1].astype(jnp.int32)
  x4 = jnp.concatenate([x.astype(_f32), jnp.zeros((N, 1), _f32)], axis=1)
  xq = x4.reshape(N * XC)

  wsrc = jnp.concatenate([W_n1[:D], W_c1[:D]], axis=1)
  wdst = jnp.concatenate([W_n1[D:2 * D], W_c1[D:2 * D]], axis=1)
  vcat = jnp.concatenate([W_n1[2 * D:], W_c1[2 * D:]], axis=1)
  bcat = jnp.concatenate([b_n1, b_c1]).reshape(1, PW)
  be1r = b_e1.reshape(1, ED)
  be2r = b_e2.reshape(1, ED)
  wc2r = W_c2.reshape(1, H)

  psrc, pdst = _proj(h, wsrc, wdst)
  zh = jnp.zeros((N, H), _f32)
  zq = jnp.zeros((N * XC,), _f32)

  srcs = [lax.slice(src, (c * EC,), ((c + 1) * EC,)) for c in range(CH)]
  dsts = [lax.slice(dst, (c * EC,), ((c + 1) * EC,)) for c in range(CH)]
  dists = [lax.slice(edge_dist, (c * EC,), ((c + 1) * EC,))
           for c in range(CH)]

  z0s = [_gather_c(srcs[c], dsts[c], psrc, pdst) for c in range(CH)]
  svw = [_edge_c(dists[c], z0s[c][0], z0s[c][1], W_e1, be1r, W_e2, be2r,
                 vcat[:, :H], vcat[:, H:],
                 b_n1.reshape(1, H), b_c1.reshape(1, H), wc2r)
         for c in range(CH)]
  hparts = [_scatter_h_c(dsts[c], svw[c][0], zh) for c in range(CH)]
  xcs = [_scatter_x_c(srcs[c], dsts[c], svw[c][1].reshape(EC), xq, zq)
         for c in range(CH)]

  xsum = _xred(jnp.concatenate(
      [xc.reshape(NW, N * XC) for xc in xcs], axis=0))
  hout, xout4 = _fold(h, x4, jnp.concatenate(hparts, axis=0),
                      xsum.reshape(N, XC), W_n2, b_n2.reshape(1, H))
  return hout, xout4[:, :3]


# speedup vs baseline: 1.4067x; 1.0974x over previous
import jax
import jax.numpy as jnp
from jax import lax
from jax.experimental import pallas as pl
from jax.experimental.pallas import tpu as pltpu
from jax.experimental.pallas import tpu_sc as plsc

N = 10000
E = 320000
D = 128
H = 128
ED = 16
XC = 4
PW = 2 * H

NC = 2
NS = 16
NW = NC * NS
L = 16
WIN = 80

CH = 1
EC = E // CH

ROWS_A = 632
ROWS_B = N - (NS - 1) * ROWS_A

_mesh = plsc.VectorSubcoreMesh(
    core_axis_name="c", subcore_axis_name="s", num_cores=NC, num_subcores=NS)

_f32 = jnp.float32
_bf16 = jnp.bfloat16


def _sig(t):
  return 1.0 / (1.0 + jnp.exp(-t))


def _rne_bf16_bits(f):
  u = lax.bitcast_convert_type(f, jnp.int32)
  u = u + 0x7FFF + (lax.shift_right_logical(u, 16) & 1)
  return lax.shift_right_logical(u, 16)


def _proj_body(h_ref, wsrc_ref, wdst_ref, psrc_ref, pdst_ref):
  hb = h_ref[:]
  for wref, pref in ((wsrc_ref, psrc_ref), (wdst_ref, pdst_ref)):
    pn = jnp.dot(hb, wref[:, :H], preferred_element_type=_f32)
    pc = jnp.dot(hb, wref[:, H:], preferred_element_type=_f32)
    pref[:] = _rne_bf16_bits(pn) | lax.shift_left(_rne_bf16_bits(pc), 16)


_proj = pl.pallas_call(
    _proj_body,
    out_shape=[jax.ShapeDtypeStruct((N, H), jnp.int32),
               jax.ShapeDtypeStruct((N, H), jnp.int32)],
)


def _make_gather(ne):
  epw = ne // NW
  nwin = epw // WIN

  def body(src_ref, dst_ref, psrc_ref, pdst_ref,
           za_ref,
           idxs_v, idxd_v, gs0, gd0, gs1, gd1,
           sem_a0, sem_b0, sem_a1, sem_b1):
    wid = lax.axis_index("s") * NC + lax.axis_index("c")
    base = wid * epw

    pltpu.sync_copy(src_ref.at[pl.ds(base, epw)], idxs_v)
    pltpu.sync_copy(dst_ref.at[pl.ds(base, epw)], idxd_v)

    def issue(w, gs, gd, sa, sb):
      sl = pl.ds(w * WIN, WIN)
      pltpu.async_copy(psrc_ref.at[idxs_v.at[sl]], gs, sa)
      pltpu.async_copy(pdst_ref.at[idxd_v.at[sl]], gd, sb)

    def process(w, gs, gd, sa, sb):
      pltpu.make_async_copy(
          psrc_ref.at[idxs_v.at[pl.ds(0, WIN)]], gs, sa).wait()
      pltpu.make_async_copy(
          pdst_ref.at[idxd_v.at[pl.ds(0, WIN)]], gd, sb).wait()

      def row(i, c2):
        for k in range(H // L):
          sl = pl.ds(k * L, L)
          a = plsc.bitcast(gs[i, sl], _bf16)
          b = plsc.bitcast(gd[i, sl], _bf16)
          gs[i, sl] = plsc.bitcast(a + b, jnp.int32)
        return c2

      lax.fori_loop(0, WIN, row, 0)
      pltpu.sync_copy(gs, za_ref.at[pl.ds(base + w * WIN, WIN)])

    issue(0, gs0, gd0, sem_a0, sem_b0)

    def pair(k, carry):
      w0 = 2 * k
      issue(w0 + 1, gs1, gd1, sem_a1, sem_b1)
      process(w0, gs0, gd0, sem_a0, sem_b0)

      @pl.when(w0 + 2 < nwin)
      def _():
        issue(w0 + 2, gs0, gd0, sem_a0, sem_b0)

      process(w0 + 1, gs1, gd1, sem_a1, sem_b1)
      return carry

    lax.fori_loop(0, nwin // 2, pair, 0)
    if nwin % 2 == 1:
      process(nwin - 1, gs0, gd0, sem_a0, sem_b0)

  return pl.kernel(
      body,
      out_type=jax.ShapeDtypeStruct((ne, H), jnp.int32),
      mesh=_mesh,
      scratch_types=[
          pltpu.VMEM((epw,), jnp.int32),
          pltpu.VMEM((epw,), jnp.int32),
          pltpu.VMEM((WIN, H), jnp.int32),
          pltpu.VMEM((WIN, H), jnp.int32),
          pltpu.VMEM((WIN, H), jnp.int32),
          pltpu.VMEM((WIN, H), jnp.int32),
          pltpu.SemaphoreType.DMA,
          pltpu.SemaphoreType.DMA,
          pltpu.SemaphoreType.DMA,
          pltpu.SemaphoreType.DMA,
      ],
      compiler_params=pltpu.CompilerParams(needs_layout_passes=False),
      cost_estimate=pl.CostEstimate(
          flops=ne * PW, transcendentals=0,
          bytes_accessed=4 * ne * H * 4),
  )


BE = 3200


def _unpack_lo(zi):
  return lax.bitcast_convert_type(lax.shift_left(zi, 16), _f32)


def _unpack_hi(zi):
  return lax.bitcast_convert_type(zi & jnp.int32(-65536), _f32)


def _make_edge(ne):
  def body(dist_ref, za_ref,
           we1_ref, be1_ref, we2_ref, be2_ref,
           vn_ref, vc_ref, bn_ref, bc_ref, wc2_ref,
           sv_ref, w_ref):
    i = pl.program_id(0)
    d = dist_ref[pl.ds(i * BE, BE)]
    e1 = d[:, None] * we1_ref[:] + be1_ref[:]
    e1 = e1 * _sig(e1)
    e2 = jnp.dot(e1, we2_ref[:], preferred_element_type=_f32) + be2_ref[:]
    za = za_ref[:]
    zn = (_unpack_lo(za)
          + jnp.dot(e2, vn_ref[:], preferred_element_type=_f32) + bn_ref[:])
    zc = (_unpack_hi(za)
          + jnp.dot(e2, vc_ref[:], preferred_element_type=_f32) + bc_ref[:])
    sv_ref[:] = zn * _sig(zn)
    s_c = zc * _sig(zc)
    s_c3 = s_c.reshape(BE // 128, 128, H)
    wc2b = jnp.broadcast_to(wc2_ref[:].reshape(1, 1, H), (BE // 128, 1, H))
    wt = lax.dot_general(wc2b, s_c3, (((2,), (2,)), ((0,), (0,))),
                         preferred_element_type=_f32)
    w_ref[:] = wt.reshape(1, BE // 128, 128)

  return pl.pallas_call(
      body,
      grid=(ne // BE,),
      in_specs=[
          pl.BlockSpec((ne,), lambda i: (0,)),
          pl.BlockSpec((BE, H), lambda i: (i, 0)),
          pl.BlockSpec((1, ED), lambda i: (0, 0)),
          pl.BlockSpec((1, ED), lambda i: (0, 0)),
          pl.BlockSpec((ED, ED), lambda i: (0, 0)),
          pl.BlockSpec((1, ED), lambda i: (0, 0)),
          pl.BlockSpec((ED, H), lambda i: (0, 0)),
          pl.BlockSpec((ED, H), lambda i: (0, 0)),
          pl.BlockSpec((1, H), lambda i: (0, 0)),
          pl.BlockSpec((1, H), lambda i: (0, 0)),
          pl.BlockSpec((1, H), lambda i: (0, 0)),
      ],
      out_specs=[
          pl.BlockSpec((BE, H), lambda i: (i, 0)),
          pl.BlockSpec((1, BE // 128, 128), lambda i: (i, 0, 0)),
      ],
      out_shape=[jax.ShapeDtypeStruct((ne, H), _f32),
                 jax.ShapeDtypeStruct((ne // BE, BE // 128, 128), _f32)],
      compiler_params=pltpu.CompilerParams(
          dimension_semantics=("arbitrary",)),
  )


def _make_scatter_h(ne):
  epw = ne // NW
  nwin = epw // WIN

  def body(dst_ref, sv_hbm, zh_ref, hpart_ref,
           idxd0, idxd1, sv0, sv1, acc_s,
           sem_i0, sem_s0, sem_i1, sem_s1):
    cid = lax.axis_index("c")
    sid = lax.axis_index("s")
    wid = sid * NC + cid
    base = wid * epw
    row0 = sid * ROWS_A

    @pl.when(sid < NS - 1)
    def _():
      pltpu.sync_copy(zh_ref.at[pl.ds(row0, ROWS_A)],
                      acc_s.at[pl.ds(row0, ROWS_A)])

    @pl.when(sid == NS - 1)
    def _():
      pltpu.sync_copy(zh_ref.at[pl.ds((NS - 1) * ROWS_A, ROWS_B)],
                      acc_s.at[pl.ds((NS - 1) * ROWS_A, ROWS_B)])

    plsc.subcore_barrier()

    def issue(w, idx_v, sv_v, si, ss):
      off = base + w * WIN
      pltpu.async_copy(dst_ref.at[pl.ds(off, WIN)], idx_v, si)
      pltpu.async_copy(sv_hbm.at[pl.ds(off, WIN)], sv_v, ss)

    def process(idx_v, sv_v, si, ss):
      pltpu.make_async_copy(dst_ref.at[pl.ds(0, WIN)], idx_v, si).wait()
      pltpu.make_async_copy(sv_hbm.at[pl.ds(0, WIN)], sv_v, ss).wait()
      pltpu.sync_copy(sv_v, acc_s.at[idx_v], add=True)

    issue(0, idxd0, sv0, sem_i0, sem_s0)

    def pair(k, carry):
      issue(2 * k + 1, idxd1, sv1, sem_i1, sem_s1)
      process(idxd0, sv0, sem_i0, sem_s0)

      @pl.when(2 * k + 2 < nwin)
      def _():
        issue(2 * k + 2, idxd0, sv0, sem_i0, sem_s0)

      process(idxd1, sv1, sem_i1, sem_s1)
      return carry

    lax.fori_loop(0, nwin // 2, pair, 0)
    if nwin % 2 == 1:
      process(idxd0, sv0, sem_i0, sem_s0)

    plsc.subcore_barrier()

    @pl.when(sid < NS - 1)
    def _():
      pltpu.sync_copy(acc_s.at[pl.ds(row0, ROWS_A)],
                      hpart_ref.at[cid, pl.ds(row0, ROWS_A)])

    @pl.when(sid == NS - 1)
    def _():
      pltpu.sync_copy(acc_s.at[pl.ds((NS - 1) * ROWS_A, ROWS_B)],
                      hpart_ref.at[cid, pl.ds((NS - 1) * ROWS_A, ROWS_B)])

  return pl.kernel(
      body,
      out_type=jax.ShapeDtypeStruct((NC, N, H), _f32),
      mesh=_mesh,
      scratch_types=[
          pltpu.VMEM((WIN,), jnp.int32),
          pltpu.VMEM((WIN,), jnp.int32),
          pltpu.VMEM((WIN, H), _f32),
          pltpu.VMEM((WIN, H), _f32),
          pltpu.VMEM_SHARED((N, H), _f32),
          pltpu.SemaphoreType.DMA,
          pltpu.SemaphoreType.DMA,
          pltpu.SemaphoreType.DMA,
          pltpu.SemaphoreType.DMA,
      ],
      cost_estimate=pl.CostEstimate(
          flops=ne * H, transcendentals=0,
          bytes_accessed=2 * ne * H * 4),
  )


def _make_scatter_x(ne):
  epw = ne // NW
  ngrp = epw // L
  rem = epw - ngrp * L
  epw_pad = epw + (L - rem if rem else 0)

  def body(src_ref, dst_ref, w_hbm, xq_hbm, zq_ref,
           xcpart_ref,
           idxs_v, idxd_v, w_v, xq_v, acc_xc):
    cid = lax.axis_index("c")
    sid = lax.axis_index("s")
    wid = sid * NC + cid
    base = wid * epw

    pltpu.sync_copy(src_ref.at[pl.ds(base, epw)], idxs_v.at[pl.ds(0, epw)])
    pltpu.sync_copy(dst_ref.at[pl.ds(base, epw)], idxd_v.at[pl.ds(0, epw)])
    pltpu.sync_copy(w_hbm.at[pl.ds(base, epw)], w_v.at[pl.ds(0, epw)])
    pltpu.sync_copy(xq_hbm, xq_v)
    pltpu.sync_copy(zq_ref, acc_xc)

    ones = jnp.full((L,), 1.0, _f32)

    def group(g, carry):
      sl = pl.ds(g * L, L)
      isrc = idxs_v[sl] * XC
      idst = idxd_v[sl] * XC
      wv = w_v[sl]
      for c in range(3):
        xs = plsc.load_gather(xq_v, [isrc + c])
        xd = plsc.load_gather(xq_v, [idst + c])
        plsc.addupdate_scatter(acc_xc, [idst + c], wv * (xs - xd))
      plsc.addupdate_scatter(acc_xc, [idst + 3], ones)
      return carry

    lax.fori_loop(0, ngrp, group, 0)

    if rem:
      sl = pl.ds(ngrp * L, L)
      mask = lax.iota(jnp.int32, L) < rem
      isrc = jnp.where(mask, idxs_v[sl], 0) * XC
      idst = jnp.where(mask, idxd_v[sl], 0) * XC
      wv = jnp.where(mask, w_v[sl], 0.0)
      for c in range(3):
        xs = plsc.load_gather(xq_v, [isrc + c])
        xd = plsc.load_gather(xq_v, [idst + c])
        plsc.addupdate_scatter(acc_xc, [idst + c], wv * (xs - xd))
      plsc.addupdate_scatter(acc_xc, [idst + 3],
                             jnp.where(mask, 1.0, 0.0).astype(_f32))

    pltpu.sync_copy(acc_xc, xcpart_ref.at[pl.ds(wid * (N * XC), N * XC)])

  return pl.kernel(
      body,
      out_type=jax.ShapeDtypeStruct((NW * N * XC,), _f32),
      mesh=_mesh,
      scratch_types=[
          pltpu.VMEM((epw_pad,), jnp.int32),
          pltpu.VMEM((epw_pad,), jnp.int32),
          pltpu.VMEM((epw_pad,), _f32),
          pltpu.VMEM((N * XC,), _f32),
          pltpu.VMEM((N * XC,), _f32),
      ],
      compiler_params=pltpu.CompilerParams(needs_layout_passes=False),
      cost_estimate=pl.CostEstimate(
          flops=8 * ne, transcendentals=0,
          bytes_accessed=3 * ne * 4 + NW * N * XC * 4),
  )


_gather_c = _make_gather(EC)
_edge_c = _make_edge(EC)
_scatter_h_c = _make_scatter_h(EC)
_scatter_x_c = _make_scatter_x(EC)


def _xred_body(xcp_ref, xsum_ref):
  xsum_ref[:] = jnp.sum(xcp_ref[:], axis=0)


_xred = pl.pallas_call(
    _xred_body,
    out_shape=jax.ShapeDtypeStruct((N * XC,), _f32),
)


def _fold_body(h_ref, x4_ref, hp_ref, xs_ref, wn2_ref, bn2_ref,
               hout_ref, xout_ref):
  hsum = hp_ref[0]
  for p in range(1, NC * CH):
    hsum = hsum + hp_ref[p]
  xsum = xs_ref[:]
  deg = xsum[:, 3:4]
  hout_ref[:] = (h_ref[:]
                 + jnp.dot(hsum, wn2_ref[:], preferred_element_type=_f32)
                 + deg * bn2_ref[:])
  xout_ref[:] = x4_ref[:] + xsum


_fold = pl.pallas_call(
    _fold_body,
    out_shape=[jax.ShapeDtypeStruct((N, H), _f32),
               jax.ShapeDtypeStruct((N, XC), _f32)],
)


def kernel(h, x, edge_index, edge_dist,
           W_e1, b_e1, W_e2, b_e2,
           W_n1, b_n1, W_n2, b_n2,
           W_c1, b_c1, W_c2):
  src = edge_index[0].astype(jnp.int32)
  dst = edge_index[1].astype(jnp.int32)
  x4 = jnp.concatenate([x.astype(_f32), jnp.zeros((N, 1), _f32)], axis=1)
  xq = x4.reshape(N * XC)

  wsrc = jnp.concatenate([W_n1[:D], W_c1[:D]], axis=1)
  wdst = jnp.concatenate([W_n1[D:2 * D], W_c1[D:2 * D]], axis=1)
  vcat = jnp.concatenate([W_n1[2 * D:], W_c1[2 * D:]], axis=1)
  bcat = jnp.concatenate([b_n1, b_c1]).reshape(1, PW)
  be1r = b_e1.reshape(1, ED)
  be2r = b_e2.reshape(1, ED)
  wc2r = W_c2.reshape(1, H)

  psrc, pdst = _proj(h, wsrc, wdst)
  zh = jnp.zeros((N, H), _f32)
  zq = jnp.zeros((N * XC,), _f32)

  srcs = [lax.slice(src, (c * EC,), ((c + 1) * EC,)) for c in range(CH)]
  dsts = [lax.slice(dst, (c * EC,), ((c + 1) * EC,)) for c in range(CH)]
  dists = [lax.slice(edge_dist, (c * EC,), ((c + 1) * EC,))
           for c in range(CH)]

  z0s = [_gather_c(srcs[c], dsts[c], psrc, pdst) for c in range(CH)]
  svw = [_edge_c(dists[c], z0s[c], W_e1, be1r, W_e2, be2r,
                 vcat[:, :H], vcat[:, H:],
                 b_n1.reshape(1, H), b_c1.reshape(1, H), wc2r)
         for c in range(CH)]
  hparts = [_scatter_h_c(dsts[c], svw[c][0], zh) for c in range(CH)]
  xcs = [_scatter_x_c(srcs[c], dsts[c], svw[c][1].reshape(EC), xq, zq)
         for c in range(CH)]

  xsum = _xred(jnp.concatenate(
      [xc.reshape(NW, N * XC) for xc in xcs], axis=0))
  hout, xout4 = _fold(h, x4, jnp.concatenate(hparts, axis=0),
                      xsum.reshape(N, XC), W_n2, b_n2.reshape(1, H))
  return hout, xout4[:, :3]


# speedup vs baseline: 1.4278x; 1.0150x over previous
import jax
import jax.numpy as jnp
from jax import lax
from jax.experimental import pallas as pl
from jax.experimental.pallas import tpu as pltpu
from jax.experimental.pallas import tpu_sc as plsc

N = 10000
E = 320000
D = 128
H = 128
ED = 16
XC = 4
PW = 2 * H

NC = 2
NS = 16
NW = NC * NS
L = 16
WIN = 80

CH = 1
EC = E // CH

ROWS_A = 632
ROWS_B = N - (NS - 1) * ROWS_A

_mesh = plsc.VectorSubcoreMesh(
    core_axis_name="c", subcore_axis_name="s", num_cores=NC, num_subcores=NS)

_f32 = jnp.float32
_bf16 = jnp.bfloat16


def _sig(t):
  return 1.0 / (1.0 + jnp.exp(-t))


def _rne_bf16_bits(f):
  u = lax.bitcast_convert_type(f, jnp.int32)
  u = u + 0x7FFF + (lax.shift_right_logical(u, 16) & 1)
  return lax.shift_right_logical(u, 16)


def _proj_body(h_ref, wsrc_ref, wdst_ref, psrc_ref, pdst_ref):
  hb = h_ref[:]
  for wref, pref in ((wsrc_ref, psrc_ref), (wdst_ref, pdst_ref)):
    pn = jnp.dot(hb, wref[:, :H], preferred_element_type=_f32)
    pc = jnp.dot(hb, wref[:, H:], preferred_element_type=_f32)
    pref[:] = _rne_bf16_bits(pn) | lax.shift_left(_rne_bf16_bits(pc), 16)


_proj = pl.pallas_call(
    _proj_body,
    out_shape=[jax.ShapeDtypeStruct((N, H), jnp.int32),
               jax.ShapeDtypeStruct((N, H), jnp.int32)],
)


def _make_gather(ne):
  epw = ne // NW
  nwin = epw // WIN

  NBUF = 4
  assert nwin % NBUF == 1

  def body(src_ref, dst_ref, psrc_ref, pdst_ref,
           za_ref,
           idxs_v, idxd_v, *bufs_and_sems):
    gs = bufs_and_sems[0:NBUF]
    gd = bufs_and_sems[NBUF:2 * NBUF]
    sa = bufs_and_sems[2 * NBUF:3 * NBUF]
    sb = bufs_and_sems[3 * NBUF:4 * NBUF]
    wid = lax.axis_index("s") * NC + lax.axis_index("c")
    base = wid * epw

    pltpu.sync_copy(src_ref.at[pl.ds(base, epw)], idxs_v)
    pltpu.sync_copy(dst_ref.at[pl.ds(base, epw)], idxd_v)

    def issue(w, j):
      sl = pl.ds(w * WIN, WIN)
      pltpu.async_copy(psrc_ref.at[idxs_v.at[sl]], gs[j], sa[j])
      pltpu.async_copy(pdst_ref.at[idxd_v.at[sl]], gd[j], sb[j])

    def process(w, j):
      pltpu.make_async_copy(
          psrc_ref.at[idxs_v.at[pl.ds(0, WIN)]], gs[j], sa[j]).wait()
      pltpu.make_async_copy(
          pdst_ref.at[idxd_v.at[pl.ds(0, WIN)]], gd[j], sb[j]).wait()

      def row(i, c2):
        for k in range(H // L):
          sl = pl.ds(k * L, L)
          a = plsc.bitcast(gs[j][i, sl], _bf16)
          b = plsc.bitcast(gd[j][i, sl], _bf16)
          gs[j][i, sl] = plsc.bitcast(a + b, jnp.int32)
        return c2

      lax.fori_loop(0, WIN, row, 0)
      pltpu.sync_copy(gs[j], za_ref.at[pl.ds(base + w * WIN, WIN)])

    for j in range(NBUF - 1):
      issue(j, j)

    def turn(k, carry):
      w0 = NBUF * k
      for j in range(NBUF):
        w = w0 + j

        @pl.when(w + NBUF - 1 < nwin)
        def _():
          issue(w + NBUF - 1, (j + NBUF - 1) % NBUF)

        process(w, j)
      return carry

    lax.fori_loop(0, nwin // NBUF, turn, 0)
    process(nwin - 1, (nwin - 1) % NBUF)

  return pl.kernel(
      body,
      out_type=jax.ShapeDtypeStruct((ne, H), jnp.int32),
      mesh=_mesh,
      scratch_types=(
          [pltpu.VMEM((epw,), jnp.int32)] * 2
          + [pltpu.VMEM((WIN, H), jnp.int32)] * (2 * NBUF)
          + [pltpu.SemaphoreType.DMA] * (2 * NBUF)
      ),
      compiler_params=pltpu.CompilerParams(needs_layout_passes=False),
      cost_estimate=pl.CostEstimate(
          flops=ne * PW, transcendentals=0,
          bytes_accessed=4 * ne * H * 4),
  )


BE = 3200


def _unpack_lo(zi):
  return lax.bitcast_convert_type(lax.shift_left(zi, 16), _f32)


def _unpack_hi(zi):
  return lax.bitcast_convert_type(zi & jnp.int32(-65536), _f32)


def _make_edge(ne):
  def body(dist_ref, za_ref,
           we1_ref, be1_ref, we2_ref, be2_ref,
           vn_ref, vc_ref, bn_ref, bc_ref, wc2_ref,
           sv_ref, w_ref):
    i = pl.program_id(0)
    d = dist_ref[pl.ds(i * BE, BE)]
    e1 = d[:, None] * we1_ref[:] + be1_ref[:]
    e1 = e1 * _sig(e1)
    e2 = jnp.dot(e1, we2_ref[:], preferred_element_type=_f32) + be2_ref[:]
    za = za_ref[:]
    zn = (_unpack_lo(za)
          + jnp.dot(e2, vn_ref[:], preferred_element_type=_f32) + bn_ref[:])
    zc = (_unpack_hi(za)
          + jnp.dot(e2, vc_ref[:], preferred_element_type=_f32) + bc_ref[:])
    sv_ref[:] = zn * _sig(zn)
    s_c = zc * _sig(zc)
    s_c3 = s_c.reshape(BE // 128, 128, H)
    wc2b = jnp.broadcast_to(wc2_ref[:].reshape(1, 1, H), (BE // 128, 1, H))
    wt = lax.dot_general(wc2b, s_c3, (((2,), (2,)), ((0,), (0,))),
                         preferred_element_type=_f32)
    w_ref[:] = wt.reshape(1, BE // 128, 128)

  return pl.pallas_call(
      body,
      grid=(ne // BE,),
      in_specs=[
          pl.BlockSpec((ne,), lambda i: (0,)),
          pl.BlockSpec((BE, H), lambda i: (i, 0)),
          pl.BlockSpec((1, ED), lambda i: (0, 0)),
          pl.BlockSpec((1, ED), lambda i: (0, 0)),
          pl.BlockSpec((ED, ED), lambda i: (0, 0)),
          pl.BlockSpec((1, ED), lambda i: (0, 0)),
          pl.BlockSpec((ED, H), lambda i: (0, 0)),
          pl.BlockSpec((ED, H), lambda i: (0, 0)),
          pl.BlockSpec((1, H), lambda i: (0, 0)),
          pl.BlockSpec((1, H), lambda i: (0, 0)),
          pl.BlockSpec((1, H), lambda i: (0, 0)),
      ],
      out_specs=[
          pl.BlockSpec((BE, H), lambda i: (i, 0)),
          pl.BlockSpec((1, BE // 128, 128), lambda i: (i, 0, 0)),
      ],
      out_shape=[jax.ShapeDtypeStruct((ne, H), _f32),
                 jax.ShapeDtypeStruct((ne // BE, BE // 128, 128), _f32)],
      compiler_params=pltpu.CompilerParams(
          dimension_semantics=("arbitrary",)),
  )


def _make_scatter_h(ne):
  epw = ne // NW
  nwin = epw // WIN

  def body(dst_ref, sv_hbm, zh_ref, hpart_ref,
           idxd0, idxd1, sv0, sv1, acc_s,
           sem_i0, sem_s0, sem_i1, sem_s1):
    cid = lax.axis_index("c")
    sid = lax.axis_index("s")
    wid = sid * NC + cid
    base = wid * epw
    row0 = sid * ROWS_A

    @pl.when(sid < NS - 1)
    def _():
      pltpu.sync_copy(zh_ref.at[pl.ds(row0, ROWS_A)],
                      acc_s.at[pl.ds(row0, ROWS_A)])

    @pl.when(sid == NS - 1)
    def _():
      pltpu.sync_copy(zh_ref.at[pl.ds((NS - 1) * ROWS_A, ROWS_B)],
                      acc_s.at[pl.ds((NS - 1) * ROWS_A, ROWS_B)])

    plsc.subcore_barrier()

    def issue(w, idx_v, sv_v, si, ss):
      off = base + w * WIN
      pltpu.async_copy(dst_ref.at[pl.ds(off, WIN)], idx_v, si)
      pltpu.async_copy(sv_hbm.at[pl.ds(off, WIN)], sv_v, ss)

    def process(idx_v, sv_v, si, ss):
      pltpu.make_async_copy(dst_ref.at[pl.ds(0, WIN)], idx_v, si).wait()
      pltpu.make_async_copy(sv_hbm.at[pl.ds(0, WIN)], sv_v, ss).wait()
      pltpu.sync_copy(sv_v, acc_s.at[idx_v], add=True)

    issue(0, idxd0, sv0, sem_i0, sem_s0)

    def pair(k, carry):
      issue(2 * k + 1, idxd1, sv1, sem_i1, sem_s1)
      process(idxd0, sv0, sem_i0, sem_s0)

      @pl.when(2 * k + 2 < nwin)
      def _():
        issue(2 * k + 2, idxd0, sv0, sem_i0, sem_s0)

      process(idxd1, sv1, sem_i1, sem_s1)
      return carry

    lax.fori_loop(0, nwin // 2, pair, 0)
    if nwin % 2 == 1:
      process(idxd0, sv0, sem_i0, sem_s0)

    plsc.subcore_barrier()

    @pl.when(sid < NS - 1)
    def _():
      pltpu.sync_copy(acc_s.at[pl.ds(row0, ROWS_A)],
                      hpart_ref.at[cid, pl.ds(row0, ROWS_A)])

    @pl.when(sid == NS - 1)
    def _():
      pltpu.sync_copy(acc_s.at[pl.ds((NS - 1) * ROWS_A, ROWS_B)],
                      hpart_ref.at[cid, pl.ds((NS - 1) * ROWS_A, ROWS_B)])

  return pl.kernel(
      body,
      out_type=jax.ShapeDtypeStruct((NC, N, H), _f32),
      mesh=_mesh,
      scratch_types=[
          pltpu.VMEM((WIN,), jnp.int32),
          pltpu.VMEM((WIN,), jnp.int32),
          pltpu.VMEM((WIN, H), _f32),
          pltpu.VMEM((WIN, H), _f32),
          pltpu.VMEM_SHARED((N, H), _f32),
          pltpu.SemaphoreType.DMA,
          pltpu.SemaphoreType.DMA,
          pltpu.SemaphoreType.DMA,
          pltpu.SemaphoreType.DMA,
      ],
      cost_estimate=pl.CostEstimate(
          flops=ne * H, transcendentals=0,
          bytes_accessed=2 * ne * H * 4),
  )


def _make_scatter_x(ne):
  epw = ne // NW
  ngrp = epw // L
  rem = epw - ngrp * L
  epw_pad = epw + (L - rem if rem else 0)

  def body(src_ref, dst_ref, w_hbm, xq_hbm, zq_ref,
           xcpart_ref,
           idxs_v, idxd_v, w_v, xq_v, acc_xc):
    cid = lax.axis_index("c")
    sid = lax.axis_index("s")
    wid = sid * NC + cid
    base = wid * epw

    pltpu.sync_copy(src_ref.at[pl.ds(base, epw)], idxs_v.at[pl.ds(0, epw)])
    pltpu.sync_copy(dst_ref.at[pl.ds(base, epw)], idxd_v.at[pl.ds(0, epw)])
    pltpu.sync_copy(w_hbm.at[pl.ds(base, epw)], w_v.at[pl.ds(0, epw)])
    pltpu.sync_copy(xq_hbm, xq_v)
    pltpu.sync_copy(zq_ref, acc_xc)

    ones = jnp.full((L,), 1.0, _f32)

    def group(g, carry):
      sl = pl.ds(g * L, L)
      isrc = idxs_v[sl] * XC
      idst = idxd_v[sl] * XC
      wv = w_v[sl]
      for c in range(3):
        xs = plsc.load_gather(xq_v, [isrc + c])
        xd = plsc.load_gather(xq_v, [idst + c])
        plsc.addupdate_scatter(acc_xc, [idst + c], wv * (xs - xd))
      plsc.addupdate_scatter(acc_xc, [idst + 3], ones)
      return carry

    lax.fori_loop(0, ngrp, group, 0)

    if rem:
      sl = pl.ds(ngrp * L, L)
      mask = lax.iota(jnp.int32, L) < rem
      isrc = jnp.where(mask, idxs_v[sl], 0) * XC
      idst = jnp.where(mask, idxd_v[sl], 0) * XC
      wv = jnp.where(mask, w_v[sl], 0.0)
      for c in range(3):
        xs = plsc.load_gather(xq_v, [isrc + c])
        xd = plsc.load_gather(xq_v, [idst + c])
        plsc.addupdate_scatter(acc_xc, [idst + c], wv * (xs - xd))
      plsc.addupdate_scatter(acc_xc, [idst + 3],
                             jnp.where(mask, 1.0, 0.0).astype(_f32))

    pltpu.sync_copy(acc_xc, xcpart_ref.at[pl.ds(wid * (N * XC), N * XC)])

  return pl.kernel(
      body,
      out_type=jax.ShapeDtypeStruct((NW * N * XC,), _f32),
      mesh=_mesh,
      scratch_types=[
          pltpu.VMEM((epw_pad,), jnp.int32),
          pltpu.VMEM((epw_pad,), jnp.int32),
          pltpu.VMEM((epw_pad,), _f32),
          pltpu.VMEM((N * XC,), _f32),
          pltpu.VMEM((N * XC,), _f32),
      ],
      compiler_params=pltpu.CompilerParams(needs_layout_passes=False),
      cost_estimate=pl.CostEstimate(
          flops=8 * ne, transcendentals=0,
          bytes_accessed=3 * ne * 4 + NW * N * XC * 4),
  )


_gather_c = _make_gather(EC)
_edge_c = _make_edge(EC)
_scatter_h_c = _make_scatter_h(EC)
_scatter_x_c = _make_scatter_x(EC)


def _xred_body(xcp_ref, xsum_ref):
  xsum_ref[:] = jnp.sum(xcp_ref[:], axis=0)


_xred = pl.pallas_call(
    _xred_body,
    out_shape=jax.ShapeDtypeStruct((N * XC,), _f32),
)


def _fold_body(h_ref, x4_ref, hp_ref, xs_ref, wn2_ref, bn2_ref,
               hout_ref, xout_ref):
  hsum = hp_ref[0]
  for p in range(1, NC * CH):
    hsum = hsum + hp_ref[p]
  xsum = xs_ref[:]
  deg = xsum[:, 3:4]
  hout_ref[:] = (h_ref[:]
                 + jnp.dot(hsum, wn2_ref[:], preferred_element_type=_f32)
                 + deg * bn2_ref[:])
  xout_ref[:] = x4_ref[:] + xsum


_fold = pl.pallas_call(
    _fold_body,
    out_shape=[jax.ShapeDtypeStruct((N, H), _f32),
               jax.ShapeDtypeStruct((N, XC), _f32)],
)


def kernel(h, x, edge_index, edge_dist,
           W_e1, b_e1, W_e2, b_e2,
           W_n1, b_n1, W_n2, b_n2,
           W_c1, b_c1, W_c2):
  src = edge_index[0].astype(jnp.int32)
  dst = edge_index[1].astype(jnp.int32)
  x4 = jnp.concatenate([x.astype(_f32), jnp.zeros((N, 1), _f32)], axis=1)
  xq = x4.reshape(N * XC)

  wsrc = jnp.concatenate([W_n1[:D], W_c1[:D]], axis=1)
  wdst = jnp.concatenate([W_n1[D:2 * D], W_c1[D:2 * D]], axis=1)
  vcat = jnp.concatenate([W_n1[2 * D:], W_c1[2 * D:]], axis=1)
  bcat = jnp.concatenate([b_n1, b_c1]).reshape(1, PW)
  be1r = b_e1.reshape(1, ED)
  be2r = b_e2.reshape(1, ED)
  wc2r = W_c2.reshape(1, H)

  psrc, pdst = _proj(h, wsrc, wdst)
  zh = jnp.zeros((N, H), _f32)
  zq = jnp.zeros((N * XC,), _f32)

  srcs = [lax.slice(src, (c * EC,), ((c + 1) * EC,)) for c in range(CH)]
  dsts = [lax.slice(dst, (c * EC,), ((c + 1) * EC,)) for c in range(CH)]
  dists = [lax.slice(edge_dist, (c * EC,), ((c + 1) * EC,))
           for c in range(CH)]

  z0s = [_gather_c(srcs[c], dsts[c], psrc, pdst) for c in range(CH)]
  svw = [_edge_c(dists[c], z0s[c], W_e1, be1r, W_e2, be2r,
                 vcat[:, :H], vcat[:, H:],
                 b_n1.reshape(1, H), b_c1.reshape(1, H), wc2r)
         for c in range(CH)]
  hparts = [_scatter_h_c(dsts[c], svw[c][0], zh) for c in range(CH)]
  xcs = [_scatter_x_c(srcs[c], dsts[c], svw[c][1].reshape(EC), xq, zq)
         for c in range(CH)]

  xsum = _xred(jnp.concatenate(
      [xc.reshape(NW, N * XC) for xc in xcs], axis=0))
  hout, xout4 = _fold(h, x4, jnp.concatenate(hparts, axis=0),
                      xsum.reshape(N, XC), W_n2, b_n2.reshape(1, H))
  return hout, xout4[:, :3]


# speedup vs baseline: 1.4804x; 1.0368x over previous
import jax
import jax.numpy as jnp
from jax import lax
from jax.experimental import pallas as pl
from jax.experimental.pallas import tpu as pltpu
from jax.experimental.pallas import tpu_sc as plsc

N = 10000
E = 320000
D = 128
H = 128
ED = 16
XC = 4
PW = 2 * H

NC = 2
NS = 16
NW = NC * NS
L = 16
WIN = 80

CH = 1
EC = E // CH

ROWS_A = 632
ROWS_B = N - (NS - 1) * ROWS_A

_mesh = plsc.VectorSubcoreMesh(
    core_axis_name="c", subcore_axis_name="s", num_cores=NC, num_subcores=NS)

_f32 = jnp.float32
_bf16 = jnp.bfloat16


def _sig(t):
  return 1.0 / (1.0 + jnp.exp(-t))


def _rne_bf16_bits(f):
  u = lax.bitcast_convert_type(f, jnp.int32)
  u = u + 0x7FFF + (lax.shift_right_logical(u, 16) & 1)
  return lax.shift_right_logical(u, 16)


def _proj_body(h_ref, wsrc_ref, wdst_ref, psrc_ref, pdst_ref):
  hb = h_ref[:]
  for wref, pref in ((wsrc_ref, psrc_ref), (wdst_ref, pdst_ref)):
    pn = jnp.dot(hb, wref[:, :H], preferred_element_type=_f32)
    pc = jnp.dot(hb, wref[:, H:], preferred_element_type=_f32)
    pref[:] = _rne_bf16_bits(pn) | lax.shift_left(_rne_bf16_bits(pc), 16)


_proj = pl.pallas_call(
    _proj_body,
    out_shape=[jax.ShapeDtypeStruct((N, H), jnp.int32),
               jax.ShapeDtypeStruct((N, H), jnp.int32)],
)


def _make_gather(ne):
  epw = ne // NW
  nwin = epw // WIN

  NBUF = 4
  assert nwin % NBUF == 1

  def body(src_ref, dst_ref, psrc_ref, pdst_ref,
           za_ref,
           idxs_v, idxd_v, *bufs_and_sems):
    gs = bufs_and_sems[0:NBUF]
    gd = bufs_and_sems[NBUF:2 * NBUF]
    sa = bufs_and_sems[2 * NBUF:3 * NBUF]
    sb = bufs_and_sems[3 * NBUF:4 * NBUF]
    wid = lax.axis_index("s") * NC + lax.axis_index("c")
    base = wid * epw

    pltpu.sync_copy(src_ref.at[pl.ds(base, epw)], idxs_v)
    pltpu.sync_copy(dst_ref.at[pl.ds(base, epw)], idxd_v)

    def issue(w, j):
      sl = pl.ds(w * WIN, WIN)
      pltpu.async_copy(psrc_ref.at[idxs_v.at[sl]], gs[j], sa[j])
      pltpu.async_copy(pdst_ref.at[idxd_v.at[sl]], gd[j], sb[j])

    def process(w, j):
      pltpu.make_async_copy(
          psrc_ref.at[idxs_v.at[pl.ds(0, WIN)]], gs[j], sa[j]).wait()
      pltpu.make_async_copy(
          pdst_ref.at[idxd_v.at[pl.ds(0, WIN)]], gd[j], sb[j]).wait()

      def row(i, c2):
        for k in range(H // L):
          sl = pl.ds(k * L, L)
          a = plsc.bitcast(gs[j][i, sl], _bf16)
          b = plsc.bitcast(gd[j][i, sl], _bf16)
          gs[j][i, sl] = plsc.bitcast(a + b, jnp.int32)
        return c2

      lax.fori_loop(0, WIN, row, 0)
      pltpu.sync_copy(gs[j], za_ref.at[pl.ds(base + w * WIN, WIN)])

    for j in range(NBUF - 1):
      issue(j, j)

    def turn(k, carry):
      w0 = NBUF * k
      for j in range(NBUF):
        w = w0 + j

        @pl.when(w + NBUF - 1 < nwin)
        def _():
          issue(w + NBUF - 1, (j + NBUF - 1) % NBUF)

        process(w, j)
      return carry

    lax.fori_loop(0, nwin // NBUF, turn, 0)
    process(nwin - 1, (nwin - 1) % NBUF)

  return pl.kernel(
      body,
      out_type=jax.ShapeDtypeStruct((ne, H), jnp.int32),
      mesh=_mesh,
      scratch_types=(
          [pltpu.VMEM((epw,), jnp.int32)] * 2
          + [pltpu.VMEM((WIN, H), jnp.int32)] * (2 * NBUF)
          + [pltpu.SemaphoreType.DMA] * (2 * NBUF)
      ),
      compiler_params=pltpu.CompilerParams(needs_layout_passes=False),
      cost_estimate=pl.CostEstimate(
          flops=ne * PW, transcendentals=0,
          bytes_accessed=4 * ne * H * 4),
  )


BE = 3200


def _unpack_lo(zi):
  return lax.bitcast_convert_type(lax.shift_left(zi, 16), _f32)


def _unpack_hi(zi):
  return lax.bitcast_convert_type(zi & jnp.int32(-65536), _f32)


def _make_edge(ne):
  def body(dist_ref, za_ref,
           we1_ref, be1_ref, we2_ref, be2_ref,
           vn_ref, vc_ref, bn_ref, bc_ref, wc2_ref,
           sv_ref, w_ref):
    i = pl.program_id(0)
    d = dist_ref[pl.ds(i * BE, BE)]
    e1 = d[:, None] * we1_ref[:] + be1_ref[:]
    e1 = e1 * _sig(e1)
    e2 = jnp.dot(e1, we2_ref[:], preferred_element_type=_f32) + be2_ref[:]
    za = za_ref[:]
    zn = (_unpack_lo(za)
          + jnp.dot(e2, vn_ref[:], preferred_element_type=_f32) + bn_ref[:])
    zc = (_unpack_hi(za)
          + jnp.dot(e2, vc_ref[:], preferred_element_type=_f32) + bc_ref[:])
    sv_ref[:] = zn * _sig(zn)
    s_c = zc * _sig(zc)
    s_c3 = s_c.reshape(BE // 128, 128, H)
    wc2b = jnp.broadcast_to(wc2_ref[:].reshape(1, 1, H), (BE // 128, 1, H))
    wt = lax.dot_general(wc2b, s_c3, (((2,), (2,)), ((0,), (0,))),
                         preferred_element_type=_f32)
    w_ref[:] = wt.reshape(1, BE // 128, 128)

  return pl.pallas_call(
      body,
      grid=(ne // BE,),
      in_specs=[
          pl.BlockSpec((ne,), lambda i: (0,)),
          pl.BlockSpec((BE, H), lambda i: (i, 0)),
          pl.BlockSpec((1, ED), lambda i: (0, 0)),
          pl.BlockSpec((1, ED), lambda i: (0, 0)),
          pl.BlockSpec((ED, ED), lambda i: (0, 0)),
          pl.BlockSpec((1, ED), lambda i: (0, 0)),
          pl.BlockSpec((ED, H), lambda i: (0, 0)),
          pl.BlockSpec((ED, H), lambda i: (0, 0)),
          pl.BlockSpec((1, H), lambda i: (0, 0)),
          pl.BlockSpec((1, H), lambda i: (0, 0)),
          pl.BlockSpec((1, H), lambda i: (0, 0)),
      ],
      out_specs=[
          pl.BlockSpec((BE, H), lambda i: (i, 0)),
          pl.BlockSpec((1, BE // 128, 128), lambda i: (i, 0, 0)),
      ],
      out_shape=[jax.ShapeDtypeStruct((ne, H), _f32),
                 jax.ShapeDtypeStruct((ne // BE, BE // 128, 128), _f32)],
      compiler_params=pltpu.CompilerParams(
          dimension_semantics=("arbitrary",)),
  )


def _make_scatter_h(ne):
  epw = ne // NW
  nwin = epw // WIN

  NBUF = 4
  assert nwin % NBUF == 1

  def body(dst_ref, sv_hbm, zh_ref, hpart_ref, *scratch):
    idxd = scratch[0:NBUF]
    sv = scratch[NBUF:2 * NBUF]
    acc_s = scratch[2 * NBUF]
    si = scratch[2 * NBUF + 1:3 * NBUF + 1]
    ss = scratch[3 * NBUF + 1:4 * NBUF + 1]
    cid = lax.axis_index("c")
    sid = lax.axis_index("s")
    wid = sid * NC + cid
    base = wid * epw
    row0 = sid * ROWS_A

    @pl.when(sid < NS - 1)
    def _():
      pltpu.sync_copy(zh_ref.at[pl.ds(row0, ROWS_A)],
                      acc_s.at[pl.ds(row0, ROWS_A)])

    @pl.when(sid == NS - 1)
    def _():
      pltpu.sync_copy(zh_ref.at[pl.ds((NS - 1) * ROWS_A, ROWS_B)],
                      acc_s.at[pl.ds((NS - 1) * ROWS_A, ROWS_B)])

    plsc.subcore_barrier()

    def issue(w, j):
      off = base + w * WIN
      pltpu.async_copy(dst_ref.at[pl.ds(off, WIN)], idxd[j], si[j])
      pltpu.async_copy(sv_hbm.at[pl.ds(off, WIN)], sv[j], ss[j])

    def process(j):
      pltpu.make_async_copy(dst_ref.at[pl.ds(0, WIN)], idxd[j], si[j]).wait()
      pltpu.make_async_copy(sv_hbm.at[pl.ds(0, WIN)], sv[j], ss[j]).wait()
      pltpu.sync_copy(sv[j], acc_s.at[idxd[j]], add=True)

    for j in range(NBUF - 1):
      issue(j, j)

    def turn(k, carry):
      w0 = NBUF * k
      for j in range(NBUF):
        w = w0 + j

        @pl.when(w + NBUF - 1 < nwin)
        def _():
          issue(w + NBUF - 1, (j + NBUF - 1) % NBUF)

        process(j)
      return carry

    lax.fori_loop(0, nwin // NBUF, turn, 0)
    process((nwin - 1) % NBUF)

    plsc.subcore_barrier()

    @pl.when(sid < NS - 1)
    def _():
      pltpu.sync_copy(acc_s.at[pl.ds(row0, ROWS_A)],
                      hpart_ref.at[cid, pl.ds(row0, ROWS_A)])

    @pl.when(sid == NS - 1)
    def _():
      pltpu.sync_copy(acc_s.at[pl.ds((NS - 1) * ROWS_A, ROWS_B)],
                      hpart_ref.at[cid, pl.ds((NS - 1) * ROWS_A, ROWS_B)])

  return pl.kernel(
      body,
      out_type=jax.ShapeDtypeStruct((NC, N, H), _f32),
      mesh=_mesh,
      scratch_types=(
          [pltpu.VMEM((WIN,), jnp.int32)] * NBUF
          + [pltpu.VMEM((WIN, H), _f32)] * NBUF
          + [pltpu.VMEM_SHARED((N, H), _f32)]
          + [pltpu.SemaphoreType.DMA] * (2 * NBUF)
      ),
      cost_estimate=pl.CostEstimate(
          flops=ne * H, transcendentals=0,
          bytes_accessed=2 * ne * H * 4),
  )


def _make_scatter_x(ne):
  epw = ne // NW
  ngrp = epw // L
  rem = epw - ngrp * L
  epw_pad = epw + (L - rem if rem else 0)

  def body(src_ref, dst_ref, w_hbm, xq_hbm, zq_ref,
           xcpart_ref,
           idxs_v, idxd_v, w_v, xq_v, acc_xc):
    cid = lax.axis_index("c")
    sid = lax.axis_index("s")
    wid = sid * NC + cid
    base = wid * epw

    pltpu.sync_copy(src_ref.at[pl.ds(base, epw)], idxs_v.at[pl.ds(0, epw)])
    pltpu.sync_copy(dst_ref.at[pl.ds(base, epw)], idxd_v.at[pl.ds(0, epw)])
    pltpu.sync_copy(w_hbm.at[pl.ds(base, epw)], w_v.at[pl.ds(0, epw)])
    pltpu.sync_copy(xq_hbm, xq_v)
    pltpu.sync_copy(zq_ref, acc_xc)

    ones = jnp.full((L,), 1.0, _f32)

    def group(g, carry):
      sl = pl.ds(g * L, L)
      isrc = idxs_v[sl] * XC
      idst = idxd_v[sl] * XC
      wv = w_v[sl]
      for c in range(3):
        xs = plsc.load_gather(xq_v, [isrc + c])
        xd = plsc.load_gather(xq_v, [idst + c])
        plsc.addupdate_scatter(acc_xc, [idst + c], wv * (xs - xd))
      plsc.addupdate_scatter(acc_xc, [idst + 3], ones)
      return carry

    lax.fori_loop(0, ngrp, group, 0)

    if rem:
      sl = pl.ds(ngrp * L, L)
      mask = lax.iota(jnp.int32, L) < rem
      isrc = jnp.where(mask, idxs_v[sl], 0) * XC
      idst = jnp.where(mask, idxd_v[sl], 0) * XC
      wv = jnp.where(mask, w_v[sl], 0.0)
      for c in range(3):
        xs = plsc.load_gather(xq_v, [isrc + c])
        xd = plsc.load_gather(xq_v, [idst + c])
        plsc.addupdate_scatter(acc_xc, [idst + c], wv * (xs - xd))
      plsc.addupdate_scatter(acc_xc, [idst + 3],
                             jnp.where(mask, 1.0, 0.0).astype(_f32))

    pltpu.sync_copy(acc_xc, xcpart_ref.at[pl.ds(wid * (N * XC), N * XC)])

  return pl.kernel(
      body,
      out_type=jax.ShapeDtypeStruct((NW * N * XC,), _f32),
      mesh=_mesh,
      scratch_types=[
          pltpu.VMEM((epw_pad,), jnp.int32),
          pltpu.VMEM((epw_pad,), jnp.int32),
          pltpu.VMEM((epw_pad,), _f32),
          pltpu.VMEM((N * XC,), _f32),
          pltpu.VMEM((N * XC,), _f32),
      ],
      compiler_params=pltpu.CompilerParams(needs_layout_passes=False),
      cost_estimate=pl.CostEstimate(
          flops=8 * ne, transcendentals=0,
          bytes_accessed=3 * ne * 4 + NW * N * XC * 4),
  )


_gather_c = _make_gather(EC)
_edge_c = _make_edge(EC)
_scatter_h_c = _make_scatter_h(EC)
_scatter_x_c = _make_scatter_x(EC)


def _xred_body(xcp_ref, xsum_ref):
  xsum_ref[:] = jnp.sum(xcp_ref[:], axis=0)


_xred = pl.pallas_call(
    _xred_body,
    out_shape=jax.ShapeDtypeStruct((N * XC,), _f32),
)


def _fold_body(h_ref, x4_ref, hp_ref, xs_ref, wn2_ref, bn2_ref,
               hout_ref, xout_ref):
  hsum = hp_ref[0]
  for p in range(1, NC * CH):
    hsum = hsum + hp_ref[p]
  xsum = xs_ref[:]
  deg = xsum[:, 3:4]
  hout_ref[:] = (h_ref[:]
                 + jnp.dot(hsum, wn2_ref[:], preferred_element_type=_f32)
                 + deg * bn2_ref[:])
  xout_ref[:] = x4_ref[:] + xsum


_fold = pl.pallas_call(
    _fold_body,
    out_shape=[jax.ShapeDtypeStruct((N, H), _f32),
               jax.ShapeDtypeStruct((N, XC), _f32)],
)


def kernel(h, x, edge_index, edge_dist,
           W_e1, b_e1, W_e2, b_e2,
           W_n1, b_n1, W_n2, b_n2,
           W_c1, b_c1, W_c2):
  src = edge_index[0].astype(jnp.int32)
  dst = edge_index[1].astype(jnp.int32)
  x4 = jnp.concatenate([x.astype(_f32), jnp.zeros((N, 1), _f32)], axis=1)
  xq = x4.reshape(N * XC)

  wsrc = jnp.concatenate([W_n1[:D], W_c1[:D]], axis=1)
  wdst = jnp.concatenate([W_n1[D:2 * D], W_c1[D:2 * D]], axis=1)
  vcat = jnp.concatenate([W_n1[2 * D:], W_c1[2 * D:]], axis=1)
  bcat = jnp.concatenate([b_n1, b_c1]).reshape(1, PW)
  be1r = b_e1.reshape(1, ED)
  be2r = b_e2.reshape(1, ED)
  wc2r = W_c2.reshape(1, H)

  psrc, pdst = _proj(h, wsrc, wdst)
  zh = jnp.zeros((N, H), _f32)
  zq = jnp.zeros((N * XC,), _f32)

  srcs = [lax.slice(src, (c * EC,), ((c + 1) * EC,)) for c in range(CH)]
  dsts = [lax.slice(dst, (c * EC,), ((c + 1) * EC,)) for c in range(CH)]
  dists = [lax.slice(edge_dist, (c * EC,), ((c + 1) * EC,))
           for c in range(CH)]

  z0s = [_gather_c(srcs[c], dsts[c], psrc, pdst) for c in range(CH)]
  svw = [_edge_c(dists[c], z0s[c], W_e1, be1r, W_e2, be2r,
                 vcat[:, :H], vcat[:, H:],
                 b_n1.reshape(1, H), b_c1.reshape(1, H), wc2r)
         for c in range(CH)]
  hparts = [_scatter_h_c(dsts[c], svw[c][0], zh) for c in range(CH)]
  xcs = [_scatter_x_c(srcs[c], dsts[c], svw[c][1].reshape(EC), xq, zq)
         for c in range(CH)]

  xsum = _xred(jnp.concatenate(
      [xc.reshape(NW, N * XC) for xc in xcs], axis=0))
  hout, xout4 = _fold(h, x4, jnp.concatenate(hparts, axis=0),
                      xsum.reshape(N, XC), W_n2, b_n2.reshape(1, H))
  return hout, xout4[:, :3]
